# Initial kernel scaffold; baseline (speedup 1.0000x reference)
import jax
import jax.numpy as jnp
from jax.experimental import pallas as pl


def kernel(x, edge_index, W1, b1, W2, b2, W3, b3):
    raise NotImplementedError("write your pallas kernel here")


# speedup vs baseline: 8.8874x; 8.8874x over previous
import functools

import jax
import jax.numpy as jnp
from jax import lax
from jax.experimental import pallas as pl
from jax.experimental.pallas import tpu as pltpu
from jax.experimental.pallas import tpu_sc as plsc

N = 10000
NP = 10240
E = 160000
CHUNK = 128
EP = 163840
NCHUNK = EP // CHUNK
NC, NS = 2, 16
RPT = NP // NS
DEGW = 128

_f32 = jnp.float32
_i32 = jnp.int32

_MESH = plsc.VectorSubcoreMesh(core_axis_name="c", subcore_axis_name="s")


@functools.partial(
    pl.kernel,
    out_type=jax.ShapeDtypeStruct((NC * NP, DEGW), _f32),
    mesh=_MESH,
    scratch_types=[
        pltpu.VMEM((CHUNK,), _i32),
        pltpu.VMEM((CHUNK, DEGW), _f32),
        pltpu.VMEM_SHARED((NP, DEGW), _f32),
    ],
)
def _deg_kernel(dst_hbm, ones_hbm, zeros_hbm, deg2_hbm, dstc, onesv, table):
    c = lax.axis_index("c")
    s = lax.axis_index("s")
    w = s * NC + c
    npc = NCHUNK // (NC * NS)
    pltpu.sync_copy(ones_hbm, onesv)
    pltpu.sync_copy(zeros_hbm.at[pl.ds(s * RPT, RPT)], table.at[pl.ds(s * RPT, RPT)])
    plsc.subcore_barrier()

    def body(j, carry):
        pltpu.sync_copy(dst_hbm.at[w * npc + j], dstc)
        pltpu.sync_copy(onesv, table.at[dstc], add=True)
        return carry

    lax.fori_loop(0, npc, body, 0)
    plsc.subcore_barrier()
    pltpu.sync_copy(table.at[pl.ds(s * RPT, RPT)],
                    deg2_hbm.at[pl.ds(c * NP + s * RPT, RPT)])


@functools.partial(
    pl.kernel,
    out_type=jax.ShapeDtypeStruct((NC * NP, 128), _f32),
    mesh=_MESH,
    scratch_types=[
        pltpu.VMEM((CHUNK,), _i32),
        pltpu.VMEM((CHUNK,), _i32),
        pltpu.VMEM((CHUNK, 128), _f32),
        pltpu.VMEM_SHARED((NP, 128), _f32),
        pltpu.SemaphoreType.DMA,
    ],
)
def _prop_kernel(g2_hbm, srccat_hbm, dst_hbm, acc2_hbm,
                 srcc, dstc, rows, acc, sem):
    c = lax.axis_index("c")
    s = lax.axis_index("s")
    npt = NCHUNK // NS
    pltpu.sync_copy(g2_hbm.at[pl.ds(c * NP + s * RPT, RPT)],
                    acc.at[pl.ds(s * RPT, RPT)])
    plsc.subcore_barrier()

    def body(j, carry):
        pltpu.sync_copy(srccat_hbm.at[c * NCHUNK + s * npt + j], srcc)
        pltpu.sync_copy(dst_hbm.at[s * npt + j], dstc)
        pltpu.async_copy(g2_hbm.at[srcc], rows, sem).wait()
        pltpu.sync_copy(rows, acc.at[dstc], add=True)
        return carry

    lax.fori_loop(0, npt, body, 0)
    plsc.subcore_barrier()
    pltpu.sync_copy(acc.at[pl.ds(s * RPT, RPT)],
                    acc2_hbm.at[pl.ds(c * NP + s * RPT, RPT)])


@functools.partial(
    pl.kernel,
    out_type=jax.ShapeDtypeStruct((NC * NP, 128), _f32),
    mesh=_MESH,
    scratch_types=[
        pltpu.VMEM((CHUNK,), _i32),
        pltpu.VMEM((CHUNK,), _i32),
        pltpu.VMEM((CHUNK, 128), _f32),
        pltpu.VMEM_SHARED((NP, 128), _f32),
        pltpu.SemaphoreType.DMA,
    ],
)
def _prop3_kernel(g_hbm, src_hbm, dst_hbm, acc2_hbm, srcc, dstc, rows, acc, sem):
    c = lax.axis_index("c")
    s = lax.axis_index("s")
    npc = NCHUNK // (NC * NS)
    base = c * (NCHUNK // NC) + s * npc
    pltpu.sync_copy(g_hbm.at[pl.ds(s * RPT, RPT)], acc.at[pl.ds(s * RPT, RPT)])
    plsc.subcore_barrier()

    def body(j, carry):
        pltpu.sync_copy(src_hbm.at[base + j], srcc)
        pltpu.sync_copy(dst_hbm.at[base + j], dstc)
        pltpu.async_copy(g_hbm.at[srcc], rows, sem).wait()
        pltpu.sync_copy(rows, acc.at[dstc], add=True)
        return carry

    lax.fori_loop(0, npc, body, 0)
    plsc.subcore_barrier()
    pltpu.sync_copy(acc.at[pl.ds(s * RPT, RPT)],
                    acc2_hbm.at[pl.ds(c * NP + s * RPT, RPT)])


_BLK = 640
_GRID = NP // _BLK

_PREC = lax.Precision.HIGHEST


def _dot(a, b):
    return lax.dot_general(a, b, (((1,), (0,)), ((), ())),
                           preferred_element_type=_f32, precision=_PREC)


def _t0_body(x_ref, w1_ref, dega_ref, degb_ref, disb_ref, g_ref):
    deg = dega_ref[:, 0:1] + degb_ref[:, 0:1] + 1.0
    dis = lax.rsqrt(deg)
    disb_ref[...] = jnp.broadcast_to(dis, (_BLK, 128))
    u = x_ref[...] * dis
    g = _dot(u, w1_ref[...])
    g_ref[0] = g[:, :128]
    g_ref[1] = g[:, 128:]


def _t1_body(acc_ref, disb_ref, b_ref, w_ref, g_ref):
    d = disb_ref[...]
    b = b_ref[...]
    zl = jnp.maximum(d * acc_ref[0] + b[:, :128], 0.0)
    zr = jnp.maximum(d * acc_ref[1] + b[:, 128:], 0.0)
    u = jnp.concatenate([d * zl, d * zr], axis=1)
    g = _dot(u, w_ref[...])
    g_ref[0] = g[:, :128]
    g_ref[1] = g[:, 128:]


def _t2_body(acc_ref, disb_ref, b_ref, w_ref, g3_ref):
    d = disb_ref[...]
    b = b_ref[...]
    zl = jnp.maximum(d * acc_ref[0] + b[:, :128], 0.0)
    zr = jnp.maximum(d * acc_ref[1] + b[:, 128:], 0.0)
    u = jnp.concatenate([d * zl, d * zr], axis=1)
    g3_ref[...] = _dot(u, w_ref[...])


def _t3_body(acc_ref, g3_ref, disb_ref, b_ref, out_ref):
    out_ref[...] = (disb_ref[...] * (acc_ref[0] + acc_ref[1] - g3_ref[...])
                    + b_ref[...])


def _row_spec(width):
    return pl.BlockSpec((_BLK, width), lambda i: (i, 0))


def _halves_spec():
    return pl.BlockSpec((2, _BLK, 128), lambda i: (0, i, 0))


def _full_spec(shape):
    return pl.BlockSpec(shape, lambda i: (0,) * len(shape))


def _t0_call(x, W1, deg2):
    return pl.pallas_call(
        _t0_body,
        grid=(_GRID,),
        in_specs=[_row_spec(256), _full_spec((256, 256)),
                  pl.BlockSpec((_BLK, DEGW), lambda i: (i, 0)),
                  pl.BlockSpec((_BLK, DEGW), lambda i: (i + NP // _BLK, 0))],
        out_specs=[_row_spec(128), _halves_spec()],
        out_shape=[jax.ShapeDtypeStruct((NP, 128), _f32),
                   jax.ShapeDtypeStruct((2, NP, 128), _f32)],
    )(x, W1, deg2, deg2)


def _t1_call(acc2, disb, b, W):
    return pl.pallas_call(
        _t1_body,
        grid=(_GRID,),
        in_specs=[_halves_spec(), _row_spec(128),
                  _full_spec((1, 256)), _full_spec((256, 256))],
        out_specs=[_halves_spec()],
        out_shape=[jax.ShapeDtypeStruct((2, NP, 128), _f32)],
    )(acc2, disb, b, W)[0]


def _t2_call(acc2, disb, b, W):
    return pl.pallas_call(
        _t2_body,
        grid=(_GRID,),
        in_specs=[_halves_spec(), _row_spec(128),
                  _full_spec((1, 256)), _full_spec((256, 128))],
        out_specs=[_row_spec(128)],
        out_shape=[jax.ShapeDtypeStruct((NP, 128), _f32)],
    )(acc2, disb, b, W)[0]


def _t3_call(acc2, g3, disb, b):
    return pl.pallas_call(
        _t3_body,
        grid=(_GRID,),
        in_specs=[_halves_spec(), _row_spec(128), _row_spec(128),
                  _full_spec((1, 128))],
        out_specs=[_row_spec(128)],
        out_shape=[jax.ShapeDtypeStruct((NP, 128), _f32)],
    )(acc2, g3, disb, b)[0]


def kernel(x, edge_index, W1, b1, W2, b2, W3, b3):
    src = edge_index[0].astype(_i32)
    dst = edge_index[1].astype(_i32)
    pad = EP - E
    api = jnp.arange(pad, dtype=_i32)
    src_p = jnp.concatenate([src, api % N])
    dst_p = jnp.concatenate([dst, N + api % (NP - N)])
    src2 = src_p.reshape(NCHUNK, CHUNK)
    dst2 = dst_p.reshape(NCHUNK, CHUNK)
    srccat = jnp.concatenate([src2, src2 + NP])

    ones_r = jnp.ones((CHUNK, DEGW), _f32)
    zdeg = jnp.zeros((NP, DEGW), _f32)
    deg2 = _deg_kernel(dst2, ones_r, zdeg)

    xp = jnp.pad(x, ((0, NP - N), (0, 0)))
    disb, g1 = _t0_call(xp, W1, deg2)
    a1 = _prop_kernel(g1.reshape(2 * NP, 128), srccat, dst2)
    g2 = _t1_call(a1.reshape(2, NP, 128), disb, b1.reshape(1, 256), W2)
    a2 = _prop_kernel(g2.reshape(2 * NP, 128), srccat, dst2)
    g3 = _t2_call(a2.reshape(2, NP, 128), disb, b2.reshape(1, 256), W3)
    a3 = _prop3_kernel(g3, src2, dst2)
    return _t3_call(a3.reshape(2, NP, 128), g3, disb, b3.reshape(1, 128))[:N]


# speedup vs baseline: 16.0189x; 1.8024x over previous
import functools

import jax
import jax.numpy as jnp
from jax import lax
from jax.experimental import pallas as pl
from jax.experimental.pallas import tpu as pltpu
from jax.experimental.pallas import tpu_sc as plsc

N = 10000
NP = 10240
E = 160000
CHUNK = 128
EP = 163840
NCHUNK = EP // CHUNK
NC, NS = 2, 16
RPT = NP // NS
DEGW = 128

_f32 = jnp.float32
_i32 = jnp.int32

_MESH = plsc.VectorSubcoreMesh(core_axis_name="c", subcore_axis_name="s")


@functools.partial(
    pl.kernel,
    out_type=jax.ShapeDtypeStruct((NC * NP, DEGW), _f32),
    mesh=_MESH,
    scratch_types=[
        pltpu.VMEM((NCHUNK // (NC * NS), CHUNK), _i32),
        pltpu.VMEM((CHUNK, DEGW), _f32),
        pltpu.VMEM_SHARED((NP, DEGW), _f32),
        pltpu.SemaphoreType.DMA,
    ],
)
def _deg_kernel(dst_hbm, ones_hbm, zeros_hbm, deg2_hbm, dstv, onesv, table, sem):
    c = lax.axis_index("c")
    s = lax.axis_index("s")
    w = s * NC + c
    npc = NCHUNK // (NC * NS)
    grp = 8
    pltpu.sync_copy(ones_hbm, onesv)
    pltpu.sync_copy(dst_hbm.at[pl.ds(w * npc, npc)], dstv)
    pltpu.sync_copy(zeros_hbm.at[pl.ds(s * RPT, RPT)], table.at[pl.ds(s * RPT, RPT)])
    plsc.subcore_barrier()

    def body(j, carry):
        for b in range(grp):
            pltpu.async_copy(onesv, table.at[dstv.at[j * grp + b]], sem, add=True)
        for b in range(grp):
            pltpu.make_async_copy(onesv, table.at[dstv.at[j * grp + b]], sem).wait()
        return carry

    lax.fori_loop(0, npc // grp, body, 0)
    plsc.subcore_barrier()
    pltpu.sync_copy(table.at[pl.ds(s * RPT, RPT)],
                    deg2_hbm.at[pl.ds(c * NP + s * RPT, RPT)])


@functools.partial(
    pl.kernel,
    out_type=jax.ShapeDtypeStruct((NC * NP, 128), _f32),
    mesh=_MESH,
    scratch_types=[
        pltpu.VMEM((NCHUNK // NS // 2, CHUNK), _i32),
        pltpu.VMEM((NCHUNK // NS // 2, CHUNK), _i32),
        pltpu.VMEM((CHUNK, 128), _f32),
        pltpu.VMEM((CHUNK, 128), _f32),
        pltpu.VMEM_SHARED((NP, 128), _f32),
        pltpu.SemaphoreType.DMA,
        pltpu.SemaphoreType.DMA,
    ],
)
def _prop_kernel(g2_hbm, srccat_hbm, dst_hbm, acc2_hbm,
                 srcv, dstv, rows0, rows1, acc, sem0, sem1):
    c = lax.axis_index("c")
    s = lax.axis_index("s")
    npt = NCHUNK // NS
    nb = npt // 2
    pltpu.sync_copy(g2_hbm.at[pl.ds(c * NP + s * RPT, RPT)],
                    acc.at[pl.ds(s * RPT, RPT)])
    plsc.subcore_barrier()

    def step(k, rows_a, sem_a, rows_b, sem_b):
        @pl.when(k + 1 < nb)
        def _():
            pltpu.async_copy(g2_hbm.at[srcv.at[k + 1]], rows_b, sem_b)
        pltpu.make_async_copy(g2_hbm.at[srcv.at[k]], rows_a, sem_a).wait()
        pltpu.sync_copy(rows_a, acc.at[dstv.at[k]], add=True)

    def body(j, carry):
        step(2 * j, rows0, sem0, rows1, sem1)
        step(2 * j + 1, rows1, sem1, rows0, sem0)
        return carry

    for m in range(2):
        pltpu.sync_copy(
            srccat_hbm.at[pl.ds(c * NCHUNK + s * npt + m * nb, nb)], srcv)
        pltpu.sync_copy(dst_hbm.at[pl.ds(s * npt + m * nb, nb)], dstv)
        pltpu.async_copy(g2_hbm.at[srcv.at[0]], rows0, sem0)
        lax.fori_loop(0, nb // 2, body, 0)

    plsc.subcore_barrier()
    pltpu.sync_copy(acc.at[pl.ds(s * RPT, RPT)],
                    acc2_hbm.at[pl.ds(c * NP + s * RPT, RPT)])


@functools.partial(
    pl.kernel,
    out_type=jax.ShapeDtypeStruct((NC * NP, 128), _f32),
    mesh=_MESH,
    scratch_types=[
        pltpu.VMEM((NCHUNK // (NC * NS), CHUNK), _i32),
        pltpu.VMEM((NCHUNK // (NC * NS), CHUNK), _i32),
        pltpu.VMEM((CHUNK, 128), _f32),
        pltpu.VMEM((CHUNK, 128), _f32),
        pltpu.VMEM_SHARED((NP, 128), _f32),
        pltpu.SemaphoreType.DMA,
        pltpu.SemaphoreType.DMA,
    ],
)
def _prop3_kernel(g_hbm, src_hbm, dst_hbm, acc2_hbm,
                  srcv, dstv, rows0, rows1, acc, sem0, sem1):
    c = lax.axis_index("c")
    s = lax.axis_index("s")
    npc = NCHUNK // (NC * NS)
    base = c * (NCHUNK // NC) + s * npc
    pltpu.sync_copy(src_hbm.at[pl.ds(base, npc)], srcv)
    pltpu.sync_copy(dst_hbm.at[pl.ds(base, npc)], dstv)
    pltpu.sync_copy(g_hbm.at[pl.ds(s * RPT, RPT)], acc.at[pl.ds(s * RPT, RPT)])
    plsc.subcore_barrier()

    pltpu.async_copy(g_hbm.at[srcv.at[0]], rows0, sem0)

    def step(k, rows_a, sem_a, rows_b, sem_b):
        @pl.when(k + 1 < npc)
        def _():
            pltpu.async_copy(g_hbm.at[srcv.at[k + 1]], rows_b, sem_b)
        pltpu.make_async_copy(g_hbm.at[srcv.at[k]], rows_a, sem_a).wait()
        pltpu.sync_copy(rows_a, acc.at[dstv.at[k]], add=True)

    def body(j, carry):
        step(2 * j, rows0, sem0, rows1, sem1)
        step(2 * j + 1, rows1, sem1, rows0, sem0)
        return carry

    lax.fori_loop(0, npc // 2, body, 0)
    plsc.subcore_barrier()
    pltpu.sync_copy(acc.at[pl.ds(s * RPT, RPT)],
                    acc2_hbm.at[pl.ds(c * NP + s * RPT, RPT)])


_BLK = 640
_GRID = NP // _BLK

_PREC = lax.Precision.HIGHEST


def _dot(a, b):
    return lax.dot_general(a, b, (((1,), (0,)), ((), ())),
                           preferred_element_type=_f32, precision=_PREC)


def _t0_body(x_ref, w1_ref, dega_ref, degb_ref, disb_ref, g_ref):
    deg = dega_ref[:, 0:1] + degb_ref[:, 0:1] + 1.0
    dis = lax.rsqrt(deg)
    disb_ref[...] = jnp.broadcast_to(dis, (_BLK, 128))
    u = x_ref[...] * dis
    g = _dot(u, w1_ref[...])
    g_ref[0] = g[:, :128]
    g_ref[1] = g[:, 128:]


def _t1_body(acc_ref, disb_ref, b_ref, w_ref, g_ref):
    d = disb_ref[...]
    b = b_ref[...]
    zl = jnp.maximum(d * acc_ref[0] + b[:, :128], 0.0)
    zr = jnp.maximum(d * acc_ref[1] + b[:, 128:], 0.0)
    u = jnp.concatenate([d * zl, d * zr], axis=1)
    g = _dot(u, w_ref[...])
    g_ref[0] = g[:, :128]
    g_ref[1] = g[:, 128:]


def _t2_body(acc_ref, disb_ref, b_ref, w_ref, g3_ref):
    d = disb_ref[...]
    b = b_ref[...]
    zl = jnp.maximum(d * acc_ref[0] + b[:, :128], 0.0)
    zr = jnp.maximum(d * acc_ref[1] + b[:, 128:], 0.0)
    u = jnp.concatenate([d * zl, d * zr], axis=1)
    g3_ref[...] = _dot(u, w_ref[...])


def _t3_body(acc_ref, g3_ref, disb_ref, b_ref, out_ref):
    out_ref[...] = (disb_ref[...] * (acc_ref[0] + acc_ref[1] - g3_ref[...])
                    + b_ref[...])


def _row_spec(width):
    return pl.BlockSpec((_BLK, width), lambda i: (i, 0))


def _halves_spec():
    return pl.BlockSpec((2, _BLK, 128), lambda i: (0, i, 0))


def _full_spec(shape):
    return pl.BlockSpec(shape, lambda i: (0,) * len(shape))


def _t0_call(x, W1, deg2):
    return pl.pallas_call(
        _t0_body,
        grid=(_GRID,),
        in_specs=[_row_spec(256), _full_spec((256, 256)),
                  pl.BlockSpec((_BLK, DEGW), lambda i: (i, 0)),
                  pl.BlockSpec((_BLK, DEGW), lambda i: (i + NP // _BLK, 0))],
        out_specs=[_row_spec(128), _halves_spec()],
        out_shape=[jax.ShapeDtypeStruct((NP, 128), _f32),
                   jax.ShapeDtypeStruct((2, NP, 128), _f32)],
    )(x, W1, deg2, deg2)


def _t1_call(acc2, disb, b, W):
    return pl.pallas_call(
        _t1_body,
        grid=(_GRID,),
        in_specs=[_halves_spec(), _row_spec(128),
                  _full_spec((1, 256)), _full_spec((256, 256))],
        out_specs=[_halves_spec()],
        out_shape=[jax.ShapeDtypeStruct((2, NP, 128), _f32)],
    )(acc2, disb, b, W)[0]


def _t2_call(acc2, disb, b, W):
    return pl.pallas_call(
        _t2_body,
        grid=(_GRID,),
        in_specs=[_halves_spec(), _row_spec(128),
                  _full_spec((1, 256)), _full_spec((256, 128))],
        out_specs=[_row_spec(128)],
        out_shape=[jax.ShapeDtypeStruct((NP, 128), _f32)],
    )(acc2, disb, b, W)[0]


def _t3_call(acc2, g3, disb, b):
    return pl.pallas_call(
        _t3_body,
        grid=(_GRID,),
        in_specs=[_halves_spec(), _row_spec(128), _row_spec(128),
                  _full_spec((1, 128))],
        out_specs=[_row_spec(128)],
        out_shape=[jax.ShapeDtypeStruct((NP, 128), _f32)],
    )(acc2, g3, disb, b)[0]


def kernel(x, edge_index, W1, b1, W2, b2, W3, b3):
    src = edge_index[0].astype(_i32)
    dst = edge_index[1].astype(_i32)
    pad = EP - E
    api = jnp.arange(pad, dtype=_i32)
    src_p = jnp.concatenate([src, api % N])
    dst_p = jnp.concatenate([dst, N + api % (NP - N)])
    src2 = src_p.reshape(NCHUNK, CHUNK)
    dst2 = dst_p.reshape(NCHUNK, CHUNK)
    srccat = jnp.concatenate([src2, src2 + NP])

    ones_r = jnp.ones((CHUNK, DEGW), _f32)
    zdeg = jnp.zeros((NP, DEGW), _f32)
    deg2 = _deg_kernel(dst2, ones_r, zdeg)

    xp = jnp.pad(x, ((0, NP - N), (0, 0)))
    disb, g1 = _t0_call(xp, W1, deg2)
    a1 = _prop_kernel(g1.reshape(2 * NP, 128), srccat, dst2)
    g2 = _t1_call(a1.reshape(2, NP, 128), disb, b1.reshape(1, 256), W2)
    a2 = _prop_kernel(g2.reshape(2 * NP, 128), srccat, dst2)
    g3 = _t2_call(a2.reshape(2, NP, 128), disb, b2.reshape(1, 256), W3)
    a3 = _prop3_kernel(g3, src2, dst2)
    return _t3_call(a3.reshape(2, NP, 128), g3, disb, b3.reshape(1, 128))[:N]


# speedup vs baseline: 16.2123x; 1.0121x over previous
import functools

import jax
import jax.numpy as jnp
from jax import lax
from jax.experimental import pallas as pl
from jax.experimental.pallas import tpu as pltpu
from jax.experimental.pallas import tpu_sc as plsc

N = 10000
NP = 10240
E = 160000
CHUNK = 128
EP = 163840
NCHUNK = EP // CHUNK
NC, NS = 2, 16
RPT = NP // NS
DEGW = 128

_f32 = jnp.float32
_i32 = jnp.int32

_MESH = plsc.VectorSubcoreMesh(core_axis_name="c", subcore_axis_name="s")


@functools.partial(
    pl.kernel,
    out_type=jax.ShapeDtypeStruct((NC * NP, DEGW), _f32),
    mesh=_MESH,
    scratch_types=[
        pltpu.VMEM((NCHUNK // (NC * NS), CHUNK), _i32),
        pltpu.VMEM((CHUNK, DEGW), _f32),
        pltpu.VMEM_SHARED((NP, DEGW), _f32),
        pltpu.SemaphoreType.DMA,
    ],
)
def _deg_kernel(dst_hbm, ones_hbm, zeros_hbm, deg2_hbm, dstv, onesv, table, sem):
    c = lax.axis_index("c")
    s = lax.axis_index("s")
    w = s * NC + c
    npc = NCHUNK // (NC * NS)
    grp = 8
    pltpu.sync_copy(ones_hbm, onesv)
    pltpu.sync_copy(dst_hbm.at[pl.ds(w * npc, npc)], dstv)
    pltpu.sync_copy(zeros_hbm.at[pl.ds(s * RPT, RPT)], table.at[pl.ds(s * RPT, RPT)])
    plsc.subcore_barrier()

    def body(j, carry):
        for b in range(grp):
            pltpu.async_copy(onesv, table.at[dstv.at[j * grp + b]], sem, add=True)
        for b in range(grp):
            pltpu.make_async_copy(onesv, table.at[dstv.at[j * grp + b]], sem).wait()
        return carry

    lax.fori_loop(0, npc // grp, body, 0)
    plsc.subcore_barrier()
    pltpu.sync_copy(table.at[pl.ds(s * RPT, RPT)],
                    deg2_hbm.at[pl.ds(c * NP + s * RPT, RPT)])


@functools.partial(
    pl.kernel,
    out_type=jax.ShapeDtypeStruct((NC * NP, 128), _f32),
    mesh=_MESH,
    scratch_types=[
        pltpu.VMEM((NCHUNK // NS // 2, CHUNK), _i32),
        pltpu.VMEM((NCHUNK // NS // 2, CHUNK), _i32),
        pltpu.VMEM((CHUNK, 128), _f32),
        pltpu.VMEM((CHUNK, 128), _f32),
        pltpu.VMEM_SHARED((NP, 128), _f32),
        pltpu.SemaphoreType.DMA,
        pltpu.SemaphoreType.DMA,
        pltpu.SemaphoreType.DMA,
        pltpu.SemaphoreType.DMA,
    ],
)
def _prop_kernel(g2_hbm, srccat_hbm, dst_hbm, acc2_hbm,
                 srcv, dstv, rows0, rows1, acc, semg0, semg1, sems0, sems1):
    c = lax.axis_index("c")
    s = lax.axis_index("s")
    npt = NCHUNK // NS
    nb = npt // 2
    pltpu.sync_copy(g2_hbm.at[pl.ds(c * NP + s * RPT, RPT)],
                    acc.at[pl.ds(s * RPT, RPT)])
    plsc.subcore_barrier()

    def gath(k, rows_x, semg_x):
        pltpu.async_copy(g2_hbm.at[srcv.at[k]], rows_x, semg_x)

    def gath_wait(k, rows_x, semg_x):
        pltpu.make_async_copy(g2_hbm.at[srcv.at[k]], rows_x, semg_x).wait()

    def scat(k, rows_x, sems_x):
        pltpu.async_copy(rows_x, acc.at[dstv.at[k]], sems_x, add=True)

    def scat_wait(k, rows_x, sems_x):
        pltpu.make_async_copy(rows_x, acc.at[dstv.at[k]], sems_x).wait()

    def step(k, rows_a, semg_a, sems_a, rows_b, semg_b, sems_b):
        @pl.when(k > 0)
        def _():
            scat_wait(k - 1, rows_b, sems_b)
        @pl.when(k + 1 < nb)
        def _():
            gath(k + 1, rows_b, semg_b)
        gath_wait(k, rows_a, semg_a)
        scat(k, rows_a, sems_a)

    def body(j, carry):
        step(2 * j, rows0, semg0, sems0, rows1, semg1, sems1)
        step(2 * j + 1, rows1, semg1, sems1, rows0, semg0, sems0)
        return carry

    for m in range(2):
        pltpu.sync_copy(
            srccat_hbm.at[pl.ds(c * NCHUNK + s * npt + m * nb, nb)], srcv)
        pltpu.sync_copy(dst_hbm.at[pl.ds(s * npt + m * nb, nb)], dstv)
        gath(0, rows0, semg0)
        lax.fori_loop(0, nb // 2, body, 0)
        scat_wait(nb - 1, rows1, sems1)

    plsc.subcore_barrier()
    pltpu.sync_copy(acc.at[pl.ds(s * RPT, RPT)],
                    acc2_hbm.at[pl.ds(c * NP + s * RPT, RPT)])


@functools.partial(
    pl.kernel,
    out_type=jax.ShapeDtypeStruct((NC * NP, 128), _f32),
    mesh=_MESH,
    scratch_types=[
        pltpu.VMEM((NCHUNK // (NC * NS), CHUNK), _i32),
        pltpu.VMEM((NCHUNK // (NC * NS), CHUNK), _i32),
        pltpu.VMEM((CHUNK, 128), _f32),
        pltpu.VMEM((CHUNK, 128), _f32),
        pltpu.VMEM_SHARED((NP, 128), _f32),
        pltpu.SemaphoreType.DMA,
        pltpu.SemaphoreType.DMA,
        pltpu.SemaphoreType.DMA,
        pltpu.SemaphoreType.DMA,
    ],
)
def _prop3_kernel(g_hbm, src_hbm, dst_hbm, acc2_hbm,
                  srcv, dstv, rows0, rows1, acc, semg0, semg1, sems0, sems1):
    c = lax.axis_index("c")
    s = lax.axis_index("s")
    npc = NCHUNK // (NC * NS)
    base = c * (NCHUNK // NC) + s * npc
    pltpu.sync_copy(src_hbm.at[pl.ds(base, npc)], srcv)
    pltpu.sync_copy(dst_hbm.at[pl.ds(base, npc)], dstv)
    pltpu.sync_copy(g_hbm.at[pl.ds(s * RPT, RPT)], acc.at[pl.ds(s * RPT, RPT)])
    plsc.subcore_barrier()

    def gath(k, rows_x, semg_x):
        pltpu.async_copy(g_hbm.at[srcv.at[k]], rows_x, semg_x)

    def gath_wait(k, rows_x, semg_x):
        pltpu.make_async_copy(g_hbm.at[srcv.at[k]], rows_x, semg_x).wait()

    def scat(k, rows_x, sems_x):
        pltpu.async_copy(rows_x, acc.at[dstv.at[k]], sems_x, add=True)

    def scat_wait(k, rows_x, sems_x):
        pltpu.make_async_copy(rows_x, acc.at[dstv.at[k]], sems_x).wait()

    def step(k, rows_a, semg_a, sems_a, rows_b, semg_b, sems_b):
        @pl.when(k > 0)
        def _():
            scat_wait(k - 1, rows_b, sems_b)
        @pl.when(k + 1 < npc)
        def _():
            gath(k + 1, rows_b, semg_b)
        gath_wait(k, rows_a, semg_a)
        scat(k, rows_a, sems_a)

    def body(j, carry):
        step(2 * j, rows0, semg0, sems0, rows1, semg1, sems1)
        step(2 * j + 1, rows1, semg1, sems1, rows0, semg0, sems0)
        return carry

    gath(0, rows0, semg0)
    lax.fori_loop(0, npc // 2, body, 0)
    scat_wait(npc - 1, rows1, sems1)
    plsc.subcore_barrier()
    pltpu.sync_copy(acc.at[pl.ds(s * RPT, RPT)],
                    acc2_hbm.at[pl.ds(c * NP + s * RPT, RPT)])


_BLK = 640
_GRID = NP // _BLK

_PREC = lax.Precision.HIGHEST


def _dot(a, b):
    return lax.dot_general(a, b, (((1,), (0,)), ((), ())),
                           preferred_element_type=_f32, precision=_PREC)


def _t0a_body(x_ref, w1_ref, p_ref):
    p = _dot(x_ref[...], w1_ref[...])
    p_ref[0] = p[:, :128]
    p_ref[1] = p[:, 128:]


def _t0b_body(p_ref, dega_ref, degb_ref, disb_ref, g_ref):
    deg = dega_ref[:, 0:1] + degb_ref[:, 0:1] + 1.0
    dis = lax.rsqrt(deg)
    d = jnp.broadcast_to(dis, (_BLK, 128))
    disb_ref[...] = d
    g_ref[0] = p_ref[0] * d
    g_ref[1] = p_ref[1] * d


def _t1_body(acc_ref, disb_ref, b_ref, w_ref, g_ref):
    d = disb_ref[...]
    b = b_ref[...]
    zl = jnp.maximum(d * acc_ref[0] + b[:, :128], 0.0)
    zr = jnp.maximum(d * acc_ref[1] + b[:, 128:], 0.0)
    u = jnp.concatenate([d * zl, d * zr], axis=1)
    g = _dot(u, w_ref[...])
    g_ref[0] = g[:, :128]
    g_ref[1] = g[:, 128:]


def _t2_body(acc_ref, disb_ref, b_ref, w_ref, g3_ref):
    d = disb_ref[...]
    b = b_ref[...]
    zl = jnp.maximum(d * acc_ref[0] + b[:, :128], 0.0)
    zr = jnp.maximum(d * acc_ref[1] + b[:, 128:], 0.0)
    u = jnp.concatenate([d * zl, d * zr], axis=1)
    g3_ref[...] = _dot(u, w_ref[...])


def _t3_body(acc_ref, g3_ref, disb_ref, b_ref, out_ref):
    out_ref[...] = (disb_ref[...] * (acc_ref[0] + acc_ref[1] - g3_ref[...])
                    + b_ref[...])


def _row_spec(width):
    return pl.BlockSpec((_BLK, width), lambda i: (i, 0))


def _halves_spec():
    return pl.BlockSpec((2, _BLK, 128), lambda i: (0, i, 0))


def _full_spec(shape):
    return pl.BlockSpec(shape, lambda i: (0,) * len(shape))


def _t0a_call(x, W1):
    return pl.pallas_call(
        _t0a_body,
        grid=(_GRID,),
        in_specs=[_row_spec(256), _full_spec((256, 256))],
        out_specs=[_halves_spec()],
        out_shape=[jax.ShapeDtypeStruct((2, NP, 128), _f32)],
    )(x, W1)[0]


def _t0b_call(p, deg2):
    return pl.pallas_call(
        _t0b_body,
        grid=(_GRID,),
        in_specs=[_halves_spec(),
                  pl.BlockSpec((_BLK, DEGW), lambda i: (i, 0)),
                  pl.BlockSpec((_BLK, DEGW), lambda i: (i + NP // _BLK, 0))],
        out_specs=[_row_spec(128), _halves_spec()],
        out_shape=[jax.ShapeDtypeStruct((NP, 128), _f32),
                   jax.ShapeDtypeStruct((2, NP, 128), _f32)],
    )(p, deg2, deg2)


def _t1_call(acc2, disb, b, W):
    return pl.pallas_call(
        _t1_body,
        grid=(_GRID,),
        in_specs=[_halves_spec(), _row_spec(128),
                  _full_spec((1, 256)), _full_spec((256, 256))],
        out_specs=[_halves_spec()],
        out_shape=[jax.ShapeDtypeStruct((2, NP, 128), _f32)],
    )(acc2, disb, b, W)[0]


def _t2_call(acc2, disb, b, W):
    return pl.pallas_call(
        _t2_body,
        grid=(_GRID,),
        in_specs=[_halves_spec(), _row_spec(128),
                  _full_spec((1, 256)), _full_spec((256, 128))],
        out_specs=[_row_spec(128)],
        out_shape=[jax.ShapeDtypeStruct((NP, 128), _f32)],
    )(acc2, disb, b, W)[0]


def _t3_call(acc2, g3, disb, b):
    return pl.pallas_call(
        _t3_body,
        grid=(_GRID,),
        in_specs=[_halves_spec(), _row_spec(128), _row_spec(128),
                  _full_spec((1, 128))],
        out_specs=[_row_spec(128)],
        out_shape=[jax.ShapeDtypeStruct((NP, 128), _f32)],
    )(acc2, g3, disb, b)[0]


def kernel(x, edge_index, W1, b1, W2, b2, W3, b3):
    src = edge_index[0].astype(_i32)
    dst = edge_index[1].astype(_i32)
    pad = EP - E
    api = jnp.arange(pad, dtype=_i32)
    src_p = jnp.concatenate([src, api % N])
    dst_p = jnp.concatenate([dst, N + api % (NP - N)])
    src2 = src_p.reshape(NCHUNK, CHUNK)
    dst2 = dst_p.reshape(NCHUNK, CHUNK)
    srccat = jnp.concatenate([src2, src2 + NP])

    ones_r = jnp.ones((CHUNK, DEGW), _f32)
    zdeg = jnp.zeros((NP, DEGW), _f32)
    deg2 = _deg_kernel(dst2, ones_r, zdeg)

    xp = jnp.pad(x, ((0, NP - N), (0, 0)))
    p1 = _t0a_call(xp, W1)
    disb, g1 = _t0b_call(p1, deg2)
    a1 = _prop_kernel(g1.reshape(2 * NP, 128), srccat, dst2)
    g2 = _t1_call(a1.reshape(2, NP, 128), disb, b1.reshape(1, 256), W2)
    a2 = _prop_kernel(g2.reshape(2 * NP, 128), srccat, dst2)
    g3 = _t2_call(a2.reshape(2, NP, 128), disb, b2.reshape(1, 256), W3)
    a3 = _prop3_kernel(g3, src2, dst2)
    return _t3_call(a3.reshape(2, NP, 128), g3, disb, b3.reshape(1, 128))[:N]


# speedup vs baseline: 16.4820x; 1.0166x over previous
import functools

import jax
import jax.numpy as jnp
from jax import lax
from jax.experimental import pallas as pl
from jax.experimental.pallas import tpu as pltpu
from jax.experimental.pallas import tpu_sc as plsc

N = 10000
NP = 10240
E = 160000
CHUNK = 128
EP = 163840
NCHUNK = EP // CHUNK
NC, NS = 2, 16
RPT = NP // NS
DEGW = 128

_f32 = jnp.float32
_i32 = jnp.int32

_MESH = plsc.VectorSubcoreMesh(core_axis_name="c", subcore_axis_name="s")


@functools.partial(
    pl.kernel,
    out_type=jax.ShapeDtypeStruct((NC * NP, DEGW), _f32),
    mesh=_MESH,
    scratch_types=[
        pltpu.VMEM((NCHUNK // (NC * NS), CHUNK), _i32),
        pltpu.VMEM((CHUNK, DEGW), _f32),
        pltpu.VMEM_SHARED((NP, DEGW), _f32),
        pltpu.SemaphoreType.DMA,
    ],
)
def _deg_kernel(dst_hbm, ones_hbm, zeros_hbm, deg2_hbm, dstv, onesv, table, sem):
    c = lax.axis_index("c")
    s = lax.axis_index("s")
    w = s * NC + c
    npc = NCHUNK // (NC * NS)
    grp = 8
    pltpu.sync_copy(ones_hbm, onesv)
    pltpu.sync_copy(dst_hbm.at[pl.ds(w * npc, npc)], dstv)
    pltpu.sync_copy(zeros_hbm.at[pl.ds(s * RPT, RPT)], table.at[pl.ds(s * RPT, RPT)])
    plsc.subcore_barrier()

    def body(j, carry):
        for b in range(grp):
            pltpu.async_copy(onesv, table.at[dstv.at[j * grp + b]], sem, add=True)
        for b in range(grp):
            pltpu.make_async_copy(onesv, table.at[dstv.at[j * grp + b]], sem).wait()
        return carry

    lax.fori_loop(0, npc // grp, body, 0)
    plsc.subcore_barrier()
    pltpu.sync_copy(table.at[pl.ds(s * RPT, RPT)],
                    deg2_hbm.at[pl.ds(c * NP + s * RPT, RPT)])


@functools.partial(
    pl.kernel,
    out_type=jax.ShapeDtypeStruct((NC * NP, 128), _f32),
    mesh=_MESH,
    scratch_types=[
        pltpu.VMEM((NCHUNK // NS // 2, CHUNK), _i32),
        pltpu.VMEM((NCHUNK // NS // 2, CHUNK), _i32),
        pltpu.VMEM((CHUNK, 128), _f32),
        pltpu.VMEM((CHUNK, 128), _f32),
        pltpu.VMEM_SHARED((NP, 128), _f32),
        pltpu.SemaphoreType.DMA,
        pltpu.SemaphoreType.DMA,
        pltpu.SemaphoreType.DMA,
        pltpu.SemaphoreType.DMA,
    ],
)
def _prop_kernel(g2_hbm, srccat_hbm, dst_hbm, acc2_hbm,
                 srcv, dstv, rows0, rows1, acc, semg0, semg1, sems0, sems1):
    c = lax.axis_index("c")
    s = lax.axis_index("s")
    npt = NCHUNK // NS
    nb = npt // 2
    pltpu.sync_copy(g2_hbm.at[pl.ds(c * NP + s * RPT, RPT)],
                    acc.at[pl.ds(s * RPT, RPT)])
    plsc.subcore_barrier()

    def gath(k, rows_x, semg_x):
        pltpu.async_copy(g2_hbm.at[srcv.at[k]], rows_x, semg_x)

    def gath_wait(k, rows_x, semg_x):
        pltpu.make_async_copy(g2_hbm.at[srcv.at[k]], rows_x, semg_x).wait()

    def scat(k, rows_x, sems_x):
        pltpu.async_copy(rows_x, acc.at[dstv.at[k]], sems_x, add=True)

    def scat_wait(k, rows_x, sems_x):
        pltpu.make_async_copy(rows_x, acc.at[dstv.at[k]], sems_x).wait()

    def step(k, rows_a, semg_a, sems_a, rows_b, semg_b, sems_b):
        @pl.when(k > 0)
        def _():
            scat_wait(k - 1, rows_b, sems_b)
        @pl.when(k + 1 < nb)
        def _():
            gath(k + 1, rows_b, semg_b)
        gath_wait(k, rows_a, semg_a)
        scat(k, rows_a, sems_a)

    def body(j, carry):
        step(2 * j, rows0, semg0, sems0, rows1, semg1, sems1)
        step(2 * j + 1, rows1, semg1, sems1, rows0, semg0, sems0)
        return carry

    for m in range(2):
        pltpu.sync_copy(
            srccat_hbm.at[pl.ds(c * NCHUNK + s * npt + m * nb, nb)], srcv)
        pltpu.sync_copy(dst_hbm.at[pl.ds(s * npt + m * nb, nb)], dstv)
        gath(0, rows0, semg0)
        lax.fori_loop(0, nb // 2, body, 0)
        scat_wait(nb - 1, rows1, sems1)

    plsc.subcore_barrier()
    pltpu.sync_copy(acc.at[pl.ds(s * RPT, RPT)],
                    acc2_hbm.at[pl.ds(c * NP + s * RPT, RPT)])


@functools.partial(
    pl.kernel,
    out_type=jax.ShapeDtypeStruct((NC * NP, 128), _f32),
    mesh=_MESH,
    scratch_types=[
        pltpu.VMEM((NCHUNK // (NC * NS), CHUNK), _i32),
        pltpu.VMEM((NCHUNK // (NC * NS), CHUNK), _i32),
        pltpu.VMEM((CHUNK, 128), _f32),
        pltpu.VMEM((CHUNK, 128), _f32),
        pltpu.VMEM_SHARED((NP, 128), _f32),
        pltpu.SemaphoreType.DMA,
        pltpu.SemaphoreType.DMA,
        pltpu.SemaphoreType.DMA,
        pltpu.SemaphoreType.DMA,
    ],
)
def _prop3_kernel(g_hbm, src_hbm, dst_hbm, acc2_hbm,
                  srcv, dstv, rows0, rows1, acc, semg0, semg1, sems0, sems1):
    c = lax.axis_index("c")
    s = lax.axis_index("s")
    npc = NCHUNK // (NC * NS)
    base = c * (NCHUNK // NC) + s * npc
    pltpu.sync_copy(src_hbm.at[pl.ds(base, npc)], srcv)
    pltpu.sync_copy(dst_hbm.at[pl.ds(base, npc)], dstv)
    pltpu.sync_copy(g_hbm.at[pl.ds(s * RPT, RPT)], acc.at[pl.ds(s * RPT, RPT)])
    plsc.subcore_barrier()

    def gath(k, rows_x, semg_x):
        pltpu.async_copy(g_hbm.at[srcv.at[k]], rows_x, semg_x)

    def gath_wait(k, rows_x, semg_x):
        pltpu.make_async_copy(g_hbm.at[srcv.at[k]], rows_x, semg_x).wait()

    def scat(k, rows_x, sems_x):
        pltpu.async_copy(rows_x, acc.at[dstv.at[k]], sems_x, add=True)

    def scat_wait(k, rows_x, sems_x):
        pltpu.make_async_copy(rows_x, acc.at[dstv.at[k]], sems_x).wait()

    def step(k, rows_a, semg_a, sems_a, rows_b, semg_b, sems_b):
        @pl.when(k > 0)
        def _():
            scat_wait(k - 1, rows_b, sems_b)
        @pl.when(k + 1 < npc)
        def _():
            gath(k + 1, rows_b, semg_b)
        gath_wait(k, rows_a, semg_a)
        scat(k, rows_a, sems_a)

    def body(j, carry):
        step(2 * j, rows0, semg0, sems0, rows1, semg1, sems1)
        step(2 * j + 1, rows1, semg1, sems1, rows0, semg0, sems0)
        return carry

    gath(0, rows0, semg0)
    lax.fori_loop(0, npc // 2, body, 0)
    scat_wait(npc - 1, rows1, sems1)
    plsc.subcore_barrier()
    pltpu.sync_copy(acc.at[pl.ds(s * RPT, RPT)],
                    acc2_hbm.at[pl.ds(c * NP + s * RPT, RPT)])


_BLK = 640
_GRID = NP // _BLK

_PREC = lax.Precision.DEFAULT


def _dot(a, b):
    return lax.dot_general(a, b, (((1,), (0,)), ((), ())),
                           preferred_element_type=_f32, precision=_PREC)


def _t0a_body(x_ref, w1_ref, p_ref):
    p = _dot(x_ref[...], w1_ref[...])
    p_ref[0] = p[:, :128]
    p_ref[1] = p[:, 128:]


def _t0b_body(p_ref, dega_ref, degb_ref, disb_ref, g_ref):
    deg = dega_ref[:, 0:1] + degb_ref[:, 0:1] + 1.0
    dis = lax.rsqrt(deg)
    d = jnp.broadcast_to(dis, (_BLK, 128))
    disb_ref[...] = d
    g_ref[0] = p_ref[0] * d
    g_ref[1] = p_ref[1] * d


def _t1_body(acc_ref, disb_ref, b_ref, w_ref, g_ref):
    d = disb_ref[...]
    b = b_ref[...]
    zl = jnp.maximum(d * acc_ref[0] + b[:, :128], 0.0)
    zr = jnp.maximum(d * acc_ref[1] + b[:, 128:], 0.0)
    u = jnp.concatenate([d * zl, d * zr], axis=1)
    g = _dot(u, w_ref[...])
    g_ref[0] = g[:, :128]
    g_ref[1] = g[:, 128:]


def _t2_body(acc_ref, disb_ref, b_ref, w_ref, g3_ref):
    d = disb_ref[...]
    b = b_ref[...]
    zl = jnp.maximum(d * acc_ref[0] + b[:, :128], 0.0)
    zr = jnp.maximum(d * acc_ref[1] + b[:, 128:], 0.0)
    u = jnp.concatenate([d * zl, d * zr], axis=1)
    g3_ref[...] = _dot(u, w_ref[...])


def _t3_body(acc_ref, g3_ref, disb_ref, b_ref, out_ref):
    out_ref[...] = (disb_ref[...] * (acc_ref[0] + acc_ref[1] - g3_ref[...])
                    + b_ref[...])


def _row_spec(width):
    return pl.BlockSpec((_BLK, width), lambda i: (i, 0))


def _halves_spec():
    return pl.BlockSpec((2, _BLK, 128), lambda i: (0, i, 0))


def _full_spec(shape):
    return pl.BlockSpec(shape, lambda i: (0,) * len(shape))


def _t0a_call(x, W1):
    return pl.pallas_call(
        _t0a_body,
        grid=(_GRID,),
        in_specs=[_row_spec(256), _full_spec((256, 256))],
        out_specs=[_halves_spec()],
        out_shape=[jax.ShapeDtypeStruct((2, NP, 128), _f32)],
    )(x, W1)[0]


def _t0b_call(p, deg2):
    return pl.pallas_call(
        _t0b_body,
        grid=(_GRID,),
        in_specs=[_halves_spec(),
                  pl.BlockSpec((_BLK, DEGW), lambda i: (i, 0)),
                  pl.BlockSpec((_BLK, DEGW), lambda i: (i + NP // _BLK, 0))],
        out_specs=[_row_spec(128), _halves_spec()],
        out_shape=[jax.ShapeDtypeStruct((NP, 128), _f32),
                   jax.ShapeDtypeStruct((2, NP, 128), _f32)],
    )(p, deg2, deg2)


def _t1_call(acc2, disb, b, W):
    return pl.pallas_call(
        _t1_body,
        grid=(_GRID,),
        in_specs=[_halves_spec(), _row_spec(128),
                  _full_spec((1, 256)), _full_spec((256, 256))],
        out_specs=[_halves_spec()],
        out_shape=[jax.ShapeDtypeStruct((2, NP, 128), _f32)],
    )(acc2, disb, b, W)[0]


def _t2_call(acc2, disb, b, W):
    return pl.pallas_call(
        _t2_body,
        grid=(_GRID,),
        in_specs=[_halves_spec(), _row_spec(128),
                  _full_spec((1, 256)), _full_spec((256, 128))],
        out_specs=[_row_spec(128)],
        out_shape=[jax.ShapeDtypeStruct((NP, 128), _f32)],
    )(acc2, disb, b, W)[0]


def _t3_call(acc2, g3, disb, b):
    return pl.pallas_call(
        _t3_body,
        grid=(_GRID,),
        in_specs=[_halves_spec(), _row_spec(128), _row_spec(128),
                  _full_spec((1, 128))],
        out_specs=[_row_spec(128)],
        out_shape=[jax.ShapeDtypeStruct((NP, 128), _f32)],
    )(acc2, g3, disb, b)[0]


def kernel(x, edge_index, W1, b1, W2, b2, W3, b3):
    src = edge_index[0].astype(_i32)
    dst = edge_index[1].astype(_i32)
    pad = EP - E
    api = jnp.arange(pad, dtype=_i32)
    src_p = jnp.concatenate([src, api % N])
    dst_p = jnp.concatenate([dst, N + api % (NP - N)])
    src2 = src_p.reshape(NCHUNK, CHUNK)
    dst2 = dst_p.reshape(NCHUNK, CHUNK)
    srccat = jnp.concatenate([src2, src2 + NP])

    ones_r = jnp.ones((CHUNK, DEGW), _f32)
    zdeg = jnp.zeros((NP, DEGW), _f32)
    deg2 = _deg_kernel(dst2, ones_r, zdeg)

    xp = jnp.pad(x, ((0, NP - N), (0, 0)))
    p1 = _t0a_call(xp, W1)
    disb, g1 = _t0b_call(p1, deg2)
    a1 = _prop_kernel(g1.reshape(2 * NP, 128), srccat, dst2)
    g2 = _t1_call(a1.reshape(2, NP, 128), disb, b1.reshape(1, 256), W2)
    a2 = _prop_kernel(g2.reshape(2 * NP, 128), srccat, dst2)
    g3 = _t2_call(a2.reshape(2, NP, 128), disb, b2.reshape(1, 256), W3)
    a3 = _prop3_kernel(g3, src2, dst2)
    return _t3_call(a3.reshape(2, NP, 128), g3, disb, b3.reshape(1, 128))[:N]


# speedup vs baseline: 16.6726x; 1.0116x over previous
import functools

import jax
import jax.numpy as jnp
from jax import lax
from jax.experimental import pallas as pl
from jax.experimental.pallas import tpu as pltpu
from jax.experimental.pallas import tpu_sc as plsc

N = 10000
NP = 10240
E = 160000
CHUNK = 128
EP = 163840
NCHUNK = EP // CHUNK
NC, NS = 2, 16
RPT = NP // NS
DEGW = 128

_f32 = jnp.float32
_i32 = jnp.int32

_MESH = plsc.VectorSubcoreMesh(core_axis_name="c", subcore_axis_name="s")


@functools.partial(
    pl.kernel,
    out_type=jax.ShapeDtypeStruct((NC * NP, DEGW), _f32),
    mesh=_MESH,
    scratch_types=[
        pltpu.VMEM((NCHUNK // (NC * NS), CHUNK), _i32),
        pltpu.VMEM((CHUNK, DEGW), _f32),
        pltpu.VMEM_SHARED((NP, DEGW), _f32),
        pltpu.SemaphoreType.DMA,
    ],
)
def _deg_kernel(dst_hbm, ones_hbm, zeros_hbm, deg2_hbm, dstv, onesv, table, sem):
    c = lax.axis_index("c")
    s = lax.axis_index("s")
    w = s * NC + c
    npc = NCHUNK // (NC * NS)
    grp = 8
    pltpu.sync_copy(ones_hbm, onesv)
    pltpu.sync_copy(dst_hbm.at[pl.ds(w * npc, npc)], dstv)
    pltpu.sync_copy(zeros_hbm.at[pl.ds(s * RPT, RPT)], table.at[pl.ds(s * RPT, RPT)])
    plsc.subcore_barrier()

    def body(j, carry):
        for b in range(grp):
            pltpu.async_copy(onesv, table.at[dstv.at[j * grp + b]], sem, add=True)
        for b in range(grp):
            pltpu.make_async_copy(onesv, table.at[dstv.at[j * grp + b]], sem).wait()
        return carry

    lax.fori_loop(0, npc // grp, body, 0)
    plsc.subcore_barrier()
    pltpu.sync_copy(table.at[pl.ds(s * RPT, RPT)],
                    deg2_hbm.at[pl.ds(c * NP + s * RPT, RPT)])


@functools.partial(
    pl.kernel,
    out_type=jax.ShapeDtypeStruct((NC * NP, 128), _f32),
    mesh=_MESH,
    scratch_types=[
        pltpu.VMEM((NCHUNK // NS // 2, CHUNK), _i32),
        pltpu.VMEM((NCHUNK // NS // 2, CHUNK), _i32),
        pltpu.VMEM((CHUNK, 128), _f32),
        pltpu.VMEM((CHUNK, 128), _f32),
        pltpu.VMEM_SHARED((NP, 128), _f32),
        pltpu.SemaphoreType.DMA,
        pltpu.SemaphoreType.DMA,
        pltpu.SemaphoreType.DMA,
        pltpu.SemaphoreType.DMA,
    ],
)
def _prop_kernel(g2_hbm, srccat_hbm, dst_hbm, acc2_hbm,
                 srcv, dstv, rows0, rows1, acc, semg0, semg1, sems0, sems1):
    c = lax.axis_index("c")
    s = lax.axis_index("s")
    npt = NCHUNK // NS
    nb = npt // 2
    pltpu.sync_copy(g2_hbm.at[pl.ds(c * NP + s * RPT, RPT)],
                    acc.at[pl.ds(s * RPT, RPT)])
    plsc.subcore_barrier()

    def gath(k, rows_x, semg_x):
        pltpu.async_copy(g2_hbm.at[srcv.at[k]], rows_x, semg_x)

    def gath_wait(k, rows_x, semg_x):
        pltpu.make_async_copy(g2_hbm.at[srcv.at[k]], rows_x, semg_x).wait()

    def scat(k, rows_x, sems_x):
        pltpu.async_copy(rows_x, acc.at[dstv.at[k]], sems_x, add=True)

    def scat_wait(k, rows_x, sems_x):
        pltpu.make_async_copy(rows_x, acc.at[dstv.at[k]], sems_x).wait()

    def step(k, rows_a, semg_a, sems_a, rows_b, semg_b, sems_b):
        @pl.when(k > 0)
        def _():
            scat_wait(k - 1, rows_b, sems_b)
        @pl.when(k + 1 < nb)
        def _():
            gath(k + 1, rows_b, semg_b)
        gath_wait(k, rows_a, semg_a)
        scat(k, rows_a, sems_a)

    def body(j, carry):
        step(2 * j, rows0, semg0, sems0, rows1, semg1, sems1)
        step(2 * j + 1, rows1, semg1, sems1, rows0, semg0, sems0)
        return carry

    for m in range(2):
        pltpu.sync_copy(
            srccat_hbm.at[pl.ds(c * NCHUNK + s * npt + m * nb, nb)], srcv)
        pltpu.sync_copy(dst_hbm.at[pl.ds(s * npt + m * nb, nb)], dstv)
        gath(0, rows0, semg0)
        lax.fori_loop(0, nb // 2, body, 0)
        scat_wait(nb - 1, rows1, sems1)

    plsc.subcore_barrier()
    pltpu.sync_copy(acc.at[pl.ds(s * RPT, RPT)],
                    acc2_hbm.at[pl.ds(c * NP + s * RPT, RPT)])


@functools.partial(
    pl.kernel,
    out_type=jax.ShapeDtypeStruct((NC * NP, 128), _f32),
    mesh=_MESH,
    scratch_types=[
        pltpu.VMEM((NCHUNK // (NC * NS), CHUNK), _i32),
        pltpu.VMEM((NCHUNK // (NC * NS), CHUNK), _i32),
        pltpu.VMEM((CHUNK, 128), _f32),
        pltpu.VMEM((CHUNK, 128), _f32),
        pltpu.VMEM_SHARED((NP, 128), _f32),
        pltpu.SemaphoreType.DMA,
        pltpu.SemaphoreType.DMA,
        pltpu.SemaphoreType.DMA,
        pltpu.SemaphoreType.DMA,
    ],
)
def _prop3_kernel(g_hbm, src_hbm, dst_hbm, acc2_hbm,
                  srcv, dstv, rows0, rows1, acc, semg0, semg1, sems0, sems1):
    c = lax.axis_index("c")
    s = lax.axis_index("s")
    npc = NCHUNK // (NC * NS)
    base = c * (NCHUNK // NC) + s * npc
    pltpu.sync_copy(src_hbm.at[pl.ds(base, npc)], srcv)
    pltpu.sync_copy(dst_hbm.at[pl.ds(base, npc)], dstv)
    pltpu.sync_copy(g_hbm.at[pl.ds(s * RPT, RPT)], acc.at[pl.ds(s * RPT, RPT)])
    plsc.subcore_barrier()

    def gath(k, rows_x, semg_x):
        pltpu.async_copy(g_hbm.at[srcv.at[k]], rows_x, semg_x)

    def gath_wait(k, rows_x, semg_x):
        pltpu.make_async_copy(g_hbm.at[srcv.at[k]], rows_x, semg_x).wait()

    def scat(k, rows_x, sems_x):
        pltpu.async_copy(rows_x, acc.at[dstv.at[k]], sems_x, add=True)

    def scat_wait(k, rows_x, sems_x):
        pltpu.make_async_copy(rows_x, acc.at[dstv.at[k]], sems_x).wait()

    def step(k, rows_a, semg_a, sems_a, rows_b, semg_b, sems_b):
        @pl.when(k > 0)
        def _():
            scat_wait(k - 1, rows_b, sems_b)
        @pl.when(k + 1 < npc)
        def _():
            gath(k + 1, rows_b, semg_b)
        gath_wait(k, rows_a, semg_a)
        scat(k, rows_a, sems_a)

    def body(j, carry):
        step(2 * j, rows0, semg0, sems0, rows1, semg1, sems1)
        step(2 * j + 1, rows1, semg1, sems1, rows0, semg0, sems0)
        return carry

    gath(0, rows0, semg0)
    lax.fori_loop(0, npc // 2, body, 0)
    scat_wait(npc - 1, rows1, sems1)
    plsc.subcore_barrier()
    pltpu.sync_copy(acc.at[pl.ds(s * RPT, RPT)],
                    acc2_hbm.at[pl.ds(c * NP + s * RPT, RPT)])


_BLK = 640
_GRID = NP // _BLK

_PREC = lax.Precision.DEFAULT


def _dot(a, b):
    return lax.dot_general(a, b, (((1,), (0,)), ((), ())),
                           preferred_element_type=_f32, precision=_PREC)


def _t0a_body(x_ref, w1_ref, p_ref):
    p = _dot(x_ref[...], w1_ref[...])
    p_ref[0] = p[:, :128]
    p_ref[1] = p[:, 128:]


def _t0b_body(p_ref, dega_ref, degb_ref, disb_ref, g_ref):
    deg = dega_ref[:, 0:1] + degb_ref[:, 0:1] + 1.0
    dis = lax.rsqrt(deg)
    d = jnp.broadcast_to(dis, (_BLK, 128))
    disb_ref[...] = d
    g_ref[0] = p_ref[0] * d
    g_ref[1] = p_ref[1] * d


def _t1_body(acc_ref, disb_ref, b_ref, w_ref, g_ref):
    d = disb_ref[...]
    b = b_ref[...]
    zl = jnp.maximum(d * acc_ref[0] + b[:, :128], 0.0)
    zr = jnp.maximum(d * acc_ref[1] + b[:, 128:], 0.0)
    u = jnp.concatenate([d * zl, d * zr], axis=1)
    g = _dot(u, w_ref[...])
    g_ref[0] = g[:, :128]
    g_ref[1] = g[:, 128:]


def _t2_body(acc_ref, disb_ref, b_ref, w_ref, g3_ref):
    d = disb_ref[...]
    b = b_ref[...]
    zl = jnp.maximum(d * acc_ref[0] + b[:, :128], 0.0)
    zr = jnp.maximum(d * acc_ref[1] + b[:, 128:], 0.0)
    u = jnp.concatenate([d * zl, d * zr], axis=1)
    g3_ref[...] = _dot(u, w_ref[...])


def _t3_body(acc_ref, g3_ref, disb_ref, b_ref, out_ref):
    out_ref[...] = (disb_ref[...] * (acc_ref[0] + acc_ref[1] - g3_ref[...])
                    + b_ref[...])


def _row_spec(width):
    return pl.BlockSpec((_BLK, width), lambda i: (i, 0))


def _halves_spec():
    return pl.BlockSpec((2, _BLK, 128), lambda i: (0, i, 0))


def _full_spec(shape):
    return pl.BlockSpec(shape, lambda i: (0,) * len(shape))


def _t0a_call(x, W1):
    return pl.pallas_call(
        _t0a_body,
        grid=(_GRID,),
        in_specs=[_row_spec(256), _full_spec((256, 256))],
        out_specs=[_halves_spec()],
        out_shape=[jax.ShapeDtypeStruct((2, NP, 128), _f32)],
    )(x, W1)[0]


def _t0b_call(p, deg2):
    return pl.pallas_call(
        _t0b_body,
        grid=(_GRID,),
        in_specs=[_halves_spec(),
                  pl.BlockSpec((_BLK, DEGW), lambda i: (i, 0)),
                  pl.BlockSpec((_BLK, DEGW), lambda i: (i + NP // _BLK, 0))],
        out_specs=[_row_spec(128), _halves_spec()],
        out_shape=[jax.ShapeDtypeStruct((NP, 128), _f32),
                   jax.ShapeDtypeStruct((2, NP, 128), _f32)],
    )(p, deg2, deg2)


def _t1_call(acc2, disb, b, W):
    return pl.pallas_call(
        _t1_body,
        grid=(_GRID,),
        in_specs=[_halves_spec(), _row_spec(128),
                  _full_spec((1, 256)), _full_spec((256, 256))],
        out_specs=[_halves_spec()],
        out_shape=[jax.ShapeDtypeStruct((2, NP, 128), _f32)],
    )(acc2, disb, b, W)[0]


def _t2_call(acc2, disb, b, W):
    return pl.pallas_call(
        _t2_body,
        grid=(_GRID,),
        in_specs=[_halves_spec(), _row_spec(128),
                  _full_spec((1, 256)), _full_spec((256, 128))],
        out_specs=[_row_spec(128)],
        out_shape=[jax.ShapeDtypeStruct((NP, 128), _f32)],
    )(acc2, disb, b, W)[0]


def _t3_call(acc2, g3, disb, b):
    return pl.pallas_call(
        _t3_body,
        grid=(_GRID,),
        in_specs=[_halves_spec(), _row_spec(128), _row_spec(128),
                  _full_spec((1, 128))],
        out_specs=[_row_spec(128)],
        out_shape=[jax.ShapeDtypeStruct((N, 128), _f32)],
    )(acc2, g3, disb, b)[0]


def kernel(x, edge_index, W1, b1, W2, b2, W3, b3):
    src = edge_index[0].astype(_i32)
    dst = edge_index[1].astype(_i32)
    pad = EP - E
    api = jnp.arange(pad, dtype=_i32)
    src_p = jnp.concatenate([src, api % N])
    dst_p = jnp.concatenate([dst, N + api % (NP - N)])
    src2 = src_p.reshape(NCHUNK, CHUNK)
    dst2 = dst_p.reshape(NCHUNK, CHUNK)
    srccat = jnp.concatenate([src2, src2 + NP])

    ones_r = jnp.ones((CHUNK, DEGW), _f32)
    zdeg = jnp.zeros((NP, DEGW), _f32)
    deg2 = _deg_kernel(dst2, ones_r, zdeg)

    xp = jnp.pad(x, ((0, NP - N), (0, 0)))
    p1 = _t0a_call(xp, W1)
    disb, g1 = _t0b_call(p1, deg2)
    a1 = _prop_kernel(g1.reshape(2 * NP, 128), srccat, dst2)
    g2 = _t1_call(a1.reshape(2, NP, 128), disb, b1.reshape(1, 256), W2)
    a2 = _prop_kernel(g2.reshape(2 * NP, 128), srccat, dst2)
    g3 = _t2_call(a2.reshape(2, NP, 128), disb, b2.reshape(1, 256), W3)
    a3 = _prop3_kernel(g3, src2, dst2)
    return _t3_call(a3.reshape(2, NP, 128), g3, disb, b3.reshape(1, 128))


# speedup vs baseline: 17.3123x; 1.0384x over previous
import functools

import jax
import jax.numpy as jnp
from jax import lax
from jax.experimental import pallas as pl
from jax.experimental.pallas import tpu as pltpu
from jax.experimental.pallas import tpu_sc as plsc

N = 10000
NP = 10240
E = 160000
CHUNK = 128
EP = 163840
NCHUNK = EP // CHUNK
NC, NS = 2, 16
RPT = NP // NS
DEGW = 128

_f32 = jnp.float32
_i32 = jnp.int32

_MESH = plsc.VectorSubcoreMesh(core_axis_name="c", subcore_axis_name="s")


@functools.partial(
    pl.kernel,
    out_type=jax.ShapeDtypeStruct((NC * NP, DEGW), _f32),
    mesh=_MESH,
    scratch_types=[
        pltpu.VMEM((NCHUNK // (NC * NS), CHUNK), _i32),
        pltpu.VMEM((CHUNK, DEGW), _f32),
        pltpu.VMEM_SHARED((NP, DEGW), _f32),
        pltpu.SemaphoreType.DMA,
    ],
)
def _deg_kernel(dst_hbm, ones_hbm, zeros_hbm, deg2_hbm, dstv, onesv, table, sem):
    c = lax.axis_index("c")
    s = lax.axis_index("s")
    w = s * NC + c
    npc = NCHUNK // (NC * NS)
    grp = 8
    pltpu.sync_copy(ones_hbm, onesv)
    pltpu.sync_copy(dst_hbm.at[pl.ds(w * npc, npc)], dstv)
    pltpu.sync_copy(zeros_hbm.at[pl.ds(s * RPT, RPT)], table.at[pl.ds(s * RPT, RPT)])
    plsc.subcore_barrier()

    def body(j, carry):
        for b in range(grp):
            pltpu.async_copy(onesv, table.at[dstv.at[j * grp + b]], sem, add=True)
        for b in range(grp):
            pltpu.make_async_copy(onesv, table.at[dstv.at[j * grp + b]], sem).wait()
        return carry

    lax.fori_loop(0, npc // grp, body, 0)
    plsc.subcore_barrier()
    pltpu.sync_copy(table.at[pl.ds(s * RPT, RPT)],
                    deg2_hbm.at[pl.ds(c * NP + s * RPT, RPT)])


@functools.partial(
    pl.kernel,
    out_type=jax.ShapeDtypeStruct((NC * NP, 128), _f32),
    mesh=_MESH,
    scratch_types=[
        pltpu.VMEM((NCHUNK // NS // 2, CHUNK), _i32),
        pltpu.VMEM((NCHUNK // NS // 2, CHUNK), _i32),
        pltpu.VMEM((CHUNK, 128), _f32),
        pltpu.VMEM((CHUNK, 128), _f32),
        pltpu.VMEM_SHARED((NP, 128), _f32),
        pltpu.SemaphoreType.DMA,
        pltpu.SemaphoreType.DMA,
        pltpu.SemaphoreType.DMA,
        pltpu.SemaphoreType.DMA,
    ],
)
def _prop_kernel(g2_hbm, srccat_hbm, dst_hbm, acc2_hbm,
                 srcv, dstv, rows0, rows1, acc, semg0, semg1, sems0, sems1):
    c = lax.axis_index("c")
    s = lax.axis_index("s")
    npt = NCHUNK // NS
    nb = npt // 2
    pltpu.sync_copy(g2_hbm.at[pl.ds(c * NP + s * RPT, RPT)],
                    acc.at[pl.ds(s * RPT, RPT)])
    plsc.subcore_barrier()

    def gath(k, rows_x, semg_x):
        pltpu.async_copy(g2_hbm.at[srcv.at[k]], rows_x, semg_x)

    def gath_wait(k, rows_x, semg_x):
        pltpu.make_async_copy(g2_hbm.at[srcv.at[k]], rows_x, semg_x).wait()

    def scat(k, rows_x, sems_x):
        pltpu.async_copy(rows_x, acc.at[dstv.at[k]], sems_x, add=True)

    def scat_wait(k, rows_x, sems_x):
        pltpu.make_async_copy(rows_x, acc.at[dstv.at[k]], sems_x).wait()

    def step(k, rows_a, semg_a, sems_a, rows_b, semg_b, sems_b):
        @pl.when(k > 0)
        def _():
            scat_wait(k - 1, rows_b, sems_b)
        @pl.when(k + 1 < nb)
        def _():
            gath(k + 1, rows_b, semg_b)
        gath_wait(k, rows_a, semg_a)
        scat(k, rows_a, sems_a)

    def body(j, carry):
        step(2 * j, rows0, semg0, sems0, rows1, semg1, sems1)
        step(2 * j + 1, rows1, semg1, sems1, rows0, semg0, sems0)
        return carry

    for m in range(2):
        pltpu.sync_copy(
            srccat_hbm.at[pl.ds(c * NCHUNK + s * npt + m * nb, nb)], srcv)
        pltpu.sync_copy(dst_hbm.at[pl.ds(s * npt + m * nb, nb)], dstv)
        gath(0, rows0, semg0)
        lax.fori_loop(0, nb // 2, body, 0)
        scat_wait(nb - 1, rows1, sems1)

    plsc.subcore_barrier()
    pltpu.sync_copy(acc.at[pl.ds(s * RPT, RPT)],
                    acc2_hbm.at[pl.ds(c * NP + s * RPT, RPT)])


@functools.partial(
    pl.kernel,
    out_type=jax.ShapeDtypeStruct((NC * NP, 128), _f32),
    mesh=_MESH,
    scratch_types=[
        pltpu.VMEM((NCHUNK // (NC * NS), CHUNK), _i32),
        pltpu.VMEM((NCHUNK // (NC * NS), CHUNK), _i32),
        pltpu.VMEM((CHUNK, 128), _f32),
        pltpu.VMEM((CHUNK, 128), _f32),
        pltpu.VMEM_SHARED((NP, 128), _f32),
        pltpu.SemaphoreType.DMA,
        pltpu.SemaphoreType.DMA,
        pltpu.SemaphoreType.DMA,
        pltpu.SemaphoreType.DMA,
    ],
)
def _prop3_kernel(g_hbm, src_hbm, dst_hbm, acc2_hbm,
                  srcv, dstv, rows0, rows1, acc, semg0, semg1, sems0, sems1):
    c = lax.axis_index("c")
    s = lax.axis_index("s")
    npc = NCHUNK // (NC * NS)
    base = c * (NCHUNK // NC) + s * npc
    pltpu.sync_copy(src_hbm.at[pl.ds(base, npc)], srcv)
    pltpu.sync_copy(dst_hbm.at[pl.ds(base, npc)], dstv)
    pltpu.sync_copy(g_hbm.at[pl.ds(s * RPT, RPT)], acc.at[pl.ds(s * RPT, RPT)])
    plsc.subcore_barrier()

    def gath(k, rows_x, semg_x):
        pltpu.async_copy(g_hbm.at[srcv.at[k]], rows_x, semg_x)

    def gath_wait(k, rows_x, semg_x):
        pltpu.make_async_copy(g_hbm.at[srcv.at[k]], rows_x, semg_x).wait()

    def scat(k, rows_x, sems_x):
        pltpu.async_copy(rows_x, acc.at[dstv.at[k]], sems_x, add=True)

    def scat_wait(k, rows_x, sems_x):
        pltpu.make_async_copy(rows_x, acc.at[dstv.at[k]], sems_x).wait()

    def step(k, rows_a, semg_a, sems_a, rows_b, semg_b, sems_b):
        @pl.when(k > 0)
        def _():
            scat_wait(k - 1, rows_b, sems_b)
        @pl.when(k + 1 < npc)
        def _():
            gath(k + 1, rows_b, semg_b)
        gath_wait(k, rows_a, semg_a)
        scat(k, rows_a, sems_a)

    def body(j, carry):
        step(2 * j, rows0, semg0, sems0, rows1, semg1, sems1)
        step(2 * j + 1, rows1, semg1, sems1, rows0, semg0, sems0)
        return carry

    gath(0, rows0, semg0)
    lax.fori_loop(0, npc // 2, body, 0)
    scat_wait(npc - 1, rows1, sems1)
    plsc.subcore_barrier()
    pltpu.sync_copy(acc.at[pl.ds(s * RPT, RPT)],
                    acc2_hbm.at[pl.ds(c * NP + s * RPT, RPT)])


_BLK = 1280
_GRID = NP // _BLK

_PREC = lax.Precision.DEFAULT


def _dot(a, b):
    return lax.dot_general(a, b, (((1,), (0,)), ((), ())),
                           preferred_element_type=_f32, precision=_PREC)


def _t0a_body(x_ref, w1_ref, p_ref):
    p = _dot(x_ref[...], w1_ref[...])
    p_ref[0] = p[:, :128]
    p_ref[1] = p[:, 128:]


def _t0b_body(p_ref, dega_ref, degb_ref, disb_ref, g_ref):
    deg = dega_ref[:, 0:1] + degb_ref[:, 0:1] + 1.0
    dis = lax.rsqrt(deg)
    d = jnp.broadcast_to(dis, (_BLK, 128))
    disb_ref[...] = d
    g_ref[0] = p_ref[0] * d
    g_ref[1] = p_ref[1] * d


def _t1_body(acc_ref, disb_ref, b_ref, w_ref, g_ref):
    d = disb_ref[...]
    b = b_ref[...]
    zl = jnp.maximum(d * acc_ref[0] + b[:, :128], 0.0)
    zr = jnp.maximum(d * acc_ref[1] + b[:, 128:], 0.0)
    u = jnp.concatenate([d * zl, d * zr], axis=1)
    g = _dot(u, w_ref[...])
    g_ref[0] = g[:, :128]
    g_ref[1] = g[:, 128:]


def _t2_body(acc_ref, disb_ref, b_ref, w_ref, g3_ref):
    d = disb_ref[...]
    b = b_ref[...]
    zl = jnp.maximum(d * acc_ref[0] + b[:, :128], 0.0)
    zr = jnp.maximum(d * acc_ref[1] + b[:, 128:], 0.0)
    u = jnp.concatenate([d * zl, d * zr], axis=1)
    g3_ref[...] = _dot(u, w_ref[...])


def _t3_body(acc_ref, g3_ref, disb_ref, b_ref, out_ref):
    out_ref[...] = (disb_ref[...] * (acc_ref[0] + acc_ref[1] - g3_ref[...])
                    + b_ref[...])


def _row_spec(width):
    return pl.BlockSpec((_BLK, width), lambda i: (i, 0))


def _halves_spec():
    return pl.BlockSpec((2, _BLK, 128), lambda i: (0, i, 0))


def _full_spec(shape):
    return pl.BlockSpec(shape, lambda i: (0,) * len(shape))


def _t0a_call(x, W1):
    return pl.pallas_call(
        _t0a_body,
        grid=(_GRID,),
        in_specs=[_row_spec(256), _full_spec((256, 256))],
        out_specs=[_halves_spec()],
        out_shape=[jax.ShapeDtypeStruct((2, NP, 128), _f32)],
    )(x, W1)[0]


def _t0b_call(p, deg2):
    return pl.pallas_call(
        _t0b_body,
        grid=(_GRID,),
        in_specs=[_halves_spec(),
                  pl.BlockSpec((_BLK, DEGW), lambda i: (i, 0)),
                  pl.BlockSpec((_BLK, DEGW), lambda i: (i + NP // _BLK, 0))],
        out_specs=[_row_spec(128), _halves_spec()],
        out_shape=[jax.ShapeDtypeStruct((NP, 128), _f32),
                   jax.ShapeDtypeStruct((2, NP, 128), _f32)],
    )(p, deg2, deg2)


def _t1_call(acc2, disb, b, W):
    return pl.pallas_call(
        _t1_body,
        grid=(_GRID,),
        in_specs=[_halves_spec(), _row_spec(128),
                  _full_spec((1, 256)), _full_spec((256, 256))],
        out_specs=[_halves_spec()],
        out_shape=[jax.ShapeDtypeStruct((2, NP, 128), _f32)],
    )(acc2, disb, b, W)[0]


def _t2_call(acc2, disb, b, W):
    return pl.pallas_call(
        _t2_body,
        grid=(_GRID,),
        in_specs=[_halves_spec(), _row_spec(128),
                  _full_spec((1, 256)), _full_spec((256, 128))],
        out_specs=[_row_spec(128)],
        out_shape=[jax.ShapeDtypeStruct((NP, 128), _f32)],
    )(acc2, disb, b, W)[0]


def _t3_call(acc2, g3, disb, b):
    return pl.pallas_call(
        _t3_body,
        grid=(_GRID,),
        in_specs=[_halves_spec(), _row_spec(128), _row_spec(128),
                  _full_spec((1, 128))],
        out_specs=[_row_spec(128)],
        out_shape=[jax.ShapeDtypeStruct((N, 128), _f32)],
    )(acc2, g3, disb, b)[0]


def kernel(x, edge_index, W1, b1, W2, b2, W3, b3):
    src = edge_index[0].astype(_i32)
    dst = edge_index[1].astype(_i32)
    pad = EP - E
    api = jnp.arange(pad, dtype=_i32)
    src_p = jnp.concatenate([src, api % N])
    dst_p = jnp.concatenate([dst, N + api % (NP - N)])
    src2 = src_p.reshape(NCHUNK, CHUNK)
    dst2 = dst_p.reshape(NCHUNK, CHUNK)
    srccat = jnp.concatenate([src2, src2 + NP])

    ones_r = jnp.ones((CHUNK, DEGW), _f32)
    zdeg = jnp.zeros((NP, DEGW), _f32)
    deg2 = _deg_kernel(dst2, ones_r, zdeg)

    xp = jnp.pad(x, ((0, NP - N), (0, 0)))
    p1 = _t0a_call(xp, W1)
    disb, g1 = _t0b_call(p1, deg2)
    a1 = _prop_kernel(g1.reshape(2 * NP, 128), srccat, dst2)
    g2 = _t1_call(a1.reshape(2, NP, 128), disb, b1.reshape(1, 256), W2)
    a2 = _prop_kernel(g2.reshape(2 * NP, 128), srccat, dst2)
    g3 = _t2_call(a2.reshape(2, NP, 128), disb, b2.reshape(1, 256), W3)
    a3 = _prop3_kernel(g3, src2, dst2)
    return _t3_call(a3.reshape(2, NP, 128), g3, disb, b3.reshape(1, 128))


# speedup vs baseline: 17.6067x; 1.0170x over previous
import functools

import jax
import jax.numpy as jnp
from jax import lax
from jax.experimental import pallas as pl
from jax.experimental.pallas import tpu as pltpu
from jax.experimental.pallas import tpu_sc as plsc

N = 10000
NP = 10240
E = 160000
CHUNK = 128
EP = 163840
NCHUNK = EP // CHUNK
NC, NS = 2, 16
RPT = NP // NS
DEGW = 128

_f32 = jnp.float32
_i32 = jnp.int32

_MESH = plsc.VectorSubcoreMesh(core_axis_name="c", subcore_axis_name="s")


@functools.partial(
    pl.kernel,
    out_type=jax.ShapeDtypeStruct((NC * NP, DEGW), _f32),
    mesh=_MESH,
    scratch_types=[
        pltpu.VMEM((NCHUNK // (NC * NS), CHUNK), _i32),
        pltpu.VMEM((CHUNK, DEGW), _f32),
        pltpu.VMEM_SHARED((NP, DEGW), _f32),
        pltpu.SemaphoreType.DMA,
    ],
)
def _deg_kernel(dst_hbm, ones_hbm, zeros_hbm, deg2_hbm, dstv, onesv, table, sem):
    c = lax.axis_index("c")
    s = lax.axis_index("s")
    w = s * NC + c
    npc = NCHUNK // (NC * NS)
    grp = 8
    pltpu.sync_copy(ones_hbm, onesv)
    pltpu.sync_copy(dst_hbm.at[pl.ds(w * npc, npc)], dstv)
    pltpu.sync_copy(zeros_hbm.at[pl.ds(s * RPT, RPT)], table.at[pl.ds(s * RPT, RPT)])
    plsc.subcore_barrier()

    def body(j, carry):
        for b in range(grp):
            pltpu.async_copy(onesv, table.at[dstv.at[j * grp + b]], sem, add=True)
        for b in range(grp):
            pltpu.make_async_copy(onesv, table.at[dstv.at[j * grp + b]], sem).wait()
        return carry

    lax.fori_loop(0, npc // grp, body, 0)
    plsc.subcore_barrier()
    pltpu.sync_copy(table.at[pl.ds(s * RPT, RPT)],
                    deg2_hbm.at[pl.ds(c * NP + s * RPT, RPT)])


@functools.partial(
    pl.kernel,
    out_type=jax.ShapeDtypeStruct((NC * NP, 128), _f32),
    mesh=_MESH,
    scratch_types=[
        pltpu.VMEM((NCHUNK // NS // 2, CHUNK), _i32),
        pltpu.VMEM((NCHUNK // NS // 2, CHUNK), _i32),
        pltpu.VMEM((CHUNK, 128), _f32),
        pltpu.VMEM((CHUNK, 128), _f32),
        pltpu.VMEM_SHARED((NP, 128), _f32),
        pltpu.SemaphoreType.DMA,
        pltpu.SemaphoreType.DMA,
        pltpu.SemaphoreType.DMA,
        pltpu.SemaphoreType.DMA,
    ],
)
def _prop_kernel(g2_hbm, srccat_hbm, dst_hbm, acc2_hbm,
                 srcv, dstv, rows0, rows1, acc, semg0, semg1, sems0, sems1):
    c = lax.axis_index("c")
    s = lax.axis_index("s")
    npt = NCHUNK // NS
    nb = npt // 2
    pltpu.sync_copy(g2_hbm.at[pl.ds(c * NP + s * RPT, RPT)],
                    acc.at[pl.ds(s * RPT, RPT)])
    plsc.subcore_barrier()

    def gath(k, rows_x, semg_x):
        pltpu.async_copy(g2_hbm.at[srcv.at[k]], rows_x, semg_x)

    def gath_wait(k, rows_x, semg_x):
        pltpu.make_async_copy(g2_hbm.at[srcv.at[k]], rows_x, semg_x).wait()

    def scat(k, rows_x, sems_x):
        pltpu.async_copy(rows_x, acc.at[dstv.at[k]], sems_x, add=True)

    def scat_wait(k, rows_x, sems_x):
        pltpu.make_async_copy(rows_x, acc.at[dstv.at[k]], sems_x).wait()

    def step(k, rows_a, semg_a, sems_a, rows_b, semg_b, sems_b):
        @pl.when(k > 0)
        def _():
            scat_wait(k - 1, rows_b, sems_b)
        @pl.when(k + 1 < nb)
        def _():
            gath(k + 1, rows_b, semg_b)
        gath_wait(k, rows_a, semg_a)
        scat(k, rows_a, sems_a)

    def body(j, carry):
        step(2 * j, rows0, semg0, sems0, rows1, semg1, sems1)
        step(2 * j + 1, rows1, semg1, sems1, rows0, semg0, sems0)
        return carry

    for m in range(2):
        pltpu.sync_copy(
            srccat_hbm.at[pl.ds(c * NCHUNK + s * npt + m * nb, nb)], srcv)
        pltpu.sync_copy(dst_hbm.at[pl.ds(s * npt + m * nb, nb)], dstv)
        gath(0, rows0, semg0)
        lax.fori_loop(0, nb // 2, body, 0)
        scat_wait(nb - 1, rows1, sems1)

    plsc.subcore_barrier()
    pltpu.sync_copy(acc.at[pl.ds(s * RPT, RPT)],
                    acc2_hbm.at[pl.ds(c * NP + s * RPT, RPT)])


@functools.partial(
    pl.kernel,
    out_type=jax.ShapeDtypeStruct((NC * NP, 128), _f32),
    mesh=_MESH,
    scratch_types=[
        pltpu.VMEM((NCHUNK // (NC * NS), CHUNK), _i32),
        pltpu.VMEM((NCHUNK // (NC * NS), CHUNK), _i32),
        pltpu.VMEM((CHUNK, 128), _f32),
        pltpu.VMEM((CHUNK, 128), _f32),
        pltpu.VMEM_SHARED((NP, 128), _f32),
        pltpu.SemaphoreType.DMA,
        pltpu.SemaphoreType.DMA,
        pltpu.SemaphoreType.DMA,
        pltpu.SemaphoreType.DMA,
    ],
)
def _prop3_kernel(g_hbm, src_hbm, dst_hbm, acc2_hbm,
                  srcv, dstv, rows0, rows1, acc, semg0, semg1, sems0, sems1):
    c = lax.axis_index("c")
    s = lax.axis_index("s")
    npc = NCHUNK // (NC * NS)
    base = c * (NCHUNK // NC) + s * npc
    pltpu.sync_copy(src_hbm.at[pl.ds(base, npc)], srcv)
    pltpu.sync_copy(dst_hbm.at[pl.ds(base, npc)], dstv)
    pltpu.sync_copy(g_hbm.at[pl.ds(s * RPT, RPT)], acc.at[pl.ds(s * RPT, RPT)])
    plsc.subcore_barrier()

    def gath(k, rows_x, semg_x):
        pltpu.async_copy(g_hbm.at[srcv.at[k]], rows_x, semg_x)

    def gath_wait(k, rows_x, semg_x):
        pltpu.make_async_copy(g_hbm.at[srcv.at[k]], rows_x, semg_x).wait()

    def scat(k, rows_x, sems_x):
        pltpu.async_copy(rows_x, acc.at[dstv.at[k]], sems_x, add=True)

    def scat_wait(k, rows_x, sems_x):
        pltpu.make_async_copy(rows_x, acc.at[dstv.at[k]], sems_x).wait()

    def step(k, rows_a, semg_a, sems_a, rows_b, semg_b, sems_b):
        @pl.when(k > 0)
        def _():
            scat_wait(k - 1, rows_b, sems_b)
        @pl.when(k + 1 < npc)
        def _():
            gath(k + 1, rows_b, semg_b)
        gath_wait(k, rows_a, semg_a)
        scat(k, rows_a, sems_a)

    def body(j, carry):
        step(2 * j, rows0, semg0, sems0, rows1, semg1, sems1)
        step(2 * j + 1, rows1, semg1, sems1, rows0, semg0, sems0)
        return carry

    gath(0, rows0, semg0)
    lax.fori_loop(0, npc // 2, body, 0)
    scat_wait(npc - 1, rows1, sems1)
    plsc.subcore_barrier()
    pltpu.sync_copy(acc.at[pl.ds(s * RPT, RPT)],
                    acc2_hbm.at[pl.ds(c * NP + s * RPT, RPT)])


_BLK = 2560
_GRID = NP // _BLK

_PREC = lax.Precision.DEFAULT


def _dot(a, b):
    return lax.dot_general(a, b, (((1,), (0,)), ((), ())),
                           preferred_element_type=_f32, precision=_PREC)


def _t0a_body(x_ref, w1_ref, p_ref):
    p = _dot(x_ref[...], w1_ref[...])
    p_ref[0] = p[:, :128]
    p_ref[1] = p[:, 128:]


def _t0b_body(p_ref, dega_ref, degb_ref, disb_ref, g_ref):
    deg = dega_ref[:, 0:1] + degb_ref[:, 0:1] + 1.0
    dis = lax.rsqrt(deg)
    d = jnp.broadcast_to(dis, (_BLK, 128))
    disb_ref[...] = d
    g_ref[0] = p_ref[0] * d
    g_ref[1] = p_ref[1] * d


def _t1_body(acc_ref, disb_ref, b_ref, w_ref, g_ref):
    d = disb_ref[...]
    b = b_ref[...]
    zl = jnp.maximum(d * acc_ref[0] + b[:, :128], 0.0)
    zr = jnp.maximum(d * acc_ref[1] + b[:, 128:], 0.0)
    u = jnp.concatenate([d * zl, d * zr], axis=1)
    g = _dot(u, w_ref[...])
    g_ref[0] = g[:, :128]
    g_ref[1] = g[:, 128:]


def _t2_body(acc_ref, disb_ref, b_ref, w_ref, g3_ref):
    d = disb_ref[...]
    b = b_ref[...]
    zl = jnp.maximum(d * acc_ref[0] + b[:, :128], 0.0)
    zr = jnp.maximum(d * acc_ref[1] + b[:, 128:], 0.0)
    u = jnp.concatenate([d * zl, d * zr], axis=1)
    g3_ref[...] = _dot(u, w_ref[...])


def _t3_body(acc_ref, g3_ref, disb_ref, b_ref, out_ref):
    out_ref[...] = (disb_ref[...] * (acc_ref[0] + acc_ref[1] - g3_ref[...])
                    + b_ref[...])


def _row_spec(width):
    return pl.BlockSpec((_BLK, width), lambda i: (i, 0))


def _halves_spec():
    return pl.BlockSpec((2, _BLK, 128), lambda i: (0, i, 0))


def _full_spec(shape):
    return pl.BlockSpec(shape, lambda i: (0,) * len(shape))


def _t0a_call(x, W1):
    return pl.pallas_call(
        _t0a_body,
        grid=(_GRID,),
        in_specs=[_row_spec(256), _full_spec((256, 256))],
        out_specs=[_halves_spec()],
        out_shape=[jax.ShapeDtypeStruct((2, NP, 128), _f32)],
    )(x, W1)[0]


def _t0b_call(p, deg2):
    return pl.pallas_call(
        _t0b_body,
        grid=(_GRID,),
        in_specs=[_halves_spec(),
                  pl.BlockSpec((_BLK, DEGW), lambda i: (i, 0)),
                  pl.BlockSpec((_BLK, DEGW), lambda i: (i + NP // _BLK, 0))],
        out_specs=[_row_spec(128), _halves_spec()],
        out_shape=[jax.ShapeDtypeStruct((NP, 128), _f32),
                   jax.ShapeDtypeStruct((2, NP, 128), _f32)],
    )(p, deg2, deg2)


def _t1_call(acc2, disb, b, W):
    return pl.pallas_call(
        _t1_body,
        grid=(_GRID,),
        in_specs=[_halves_spec(), _row_spec(128),
                  _full_spec((1, 256)), _full_spec((256, 256))],
        out_specs=[_halves_spec()],
        out_shape=[jax.ShapeDtypeStruct((2, NP, 128), _f32)],
    )(acc2, disb, b, W)[0]


def _t2_call(acc2, disb, b, W):
    return pl.pallas_call(
        _t2_body,
        grid=(_GRID,),
        in_specs=[_halves_spec(), _row_spec(128),
                  _full_spec((1, 256)), _full_spec((256, 128))],
        out_specs=[_row_spec(128)],
        out_shape=[jax.ShapeDtypeStruct((NP, 128), _f32)],
    )(acc2, disb, b, W)[0]


def _t3_call(acc2, g3, disb, b):
    return pl.pallas_call(
        _t3_body,
        grid=(_GRID,),
        in_specs=[_halves_spec(), _row_spec(128), _row_spec(128),
                  _full_spec((1, 128))],
        out_specs=[_row_spec(128)],
        out_shape=[jax.ShapeDtypeStruct((N, 128), _f32)],
    )(acc2, g3, disb, b)[0]


def kernel(x, edge_index, W1, b1, W2, b2, W3, b3):
    src = edge_index[0].astype(_i32)
    dst = edge_index[1].astype(_i32)
    pad = EP - E
    api = jnp.arange(pad, dtype=_i32)
    src_p = jnp.concatenate([src, api % N])
    dst_p = jnp.concatenate([dst, N + api % (NP - N)])
    src2 = src_p.reshape(NCHUNK, CHUNK)
    dst2 = dst_p.reshape(NCHUNK, CHUNK)
    srccat = jnp.concatenate([src2, src2 + NP])

    ones_r = jnp.ones((CHUNK, DEGW), _f32)
    zdeg = jnp.zeros((NP, DEGW), _f32)
    deg2 = _deg_kernel(dst2, ones_r, zdeg)

    xp = jnp.pad(x, ((0, NP - N), (0, 0)))
    p1 = _t0a_call(xp, W1)
    disb, g1 = _t0b_call(p1, deg2)
    a1 = _prop_kernel(g1.reshape(2 * NP, 128), srccat, dst2)
    g2 = _t1_call(a1.reshape(2, NP, 128), disb, b1.reshape(1, 256), W2)
    a2 = _prop_kernel(g2.reshape(2 * NP, 128), srccat, dst2)
    g3 = _t2_call(a2.reshape(2, NP, 128), disb, b2.reshape(1, 256), W3)
    a3 = _prop3_kernel(g3, src2, dst2)
    return _t3_call(a3.reshape(2, NP, 128), g3, disb, b3.reshape(1, 128))


# speedup vs baseline: 17.7235x; 1.0066x over previous
import functools

import jax
import jax.numpy as jnp
from jax import lax
from jax.experimental import pallas as pl
from jax.experimental.pallas import tpu as pltpu
from jax.experimental.pallas import tpu_sc as plsc

N = 10000
NP = 10240
E = 160000
CHUNK = 128
EP = 163840
NCHUNK = EP // CHUNK
NC, NS = 2, 16
RPT = NP // NS
DEGW = 128

_f32 = jnp.float32
_i32 = jnp.int32

_MESH = plsc.VectorSubcoreMesh(core_axis_name="c", subcore_axis_name="s")


@functools.partial(
    pl.kernel,
    out_type=jax.ShapeDtypeStruct((NC * NP, DEGW), _f32),
    mesh=_MESH,
    scratch_types=[
        pltpu.VMEM((NCHUNK // (NC * NS), CHUNK), _i32),
        pltpu.VMEM((CHUNK, DEGW), _f32),
        pltpu.VMEM_SHARED((NP, DEGW), _f32),
        pltpu.SemaphoreType.DMA,
    ],
)
def _deg_kernel(dst_hbm, ones_hbm, zeros_hbm, deg2_hbm, dstv, onesv, table, sem):
    c = lax.axis_index("c")
    s = lax.axis_index("s")
    w = s * NC + c
    npc = NCHUNK // (NC * NS)
    grp = 8
    pltpu.sync_copy(ones_hbm, onesv)
    pltpu.sync_copy(dst_hbm.at[pl.ds(w * npc, npc)], dstv)
    pltpu.sync_copy(zeros_hbm.at[pl.ds(s * RPT, RPT)], table.at[pl.ds(s * RPT, RPT)])
    plsc.subcore_barrier()

    def body(j, carry):
        for b in range(grp):
            pltpu.async_copy(onesv, table.at[dstv.at[j * grp + b]], sem, add=True)
        for b in range(grp):
            pltpu.make_async_copy(onesv, table.at[dstv.at[j * grp + b]], sem).wait()
        return carry

    lax.fori_loop(0, npc // grp, body, 0)
    plsc.subcore_barrier()
    pltpu.sync_copy(table.at[pl.ds(s * RPT, RPT)],
                    deg2_hbm.at[pl.ds(c * NP + s * RPT, RPT)])


@functools.partial(
    pl.kernel,
    out_type=jax.ShapeDtypeStruct((NC * NP, 128), _f32),
    mesh=_MESH,
    scratch_types=[
        pltpu.VMEM((NCHUNK // NS // 2, CHUNK), _i32),
        pltpu.VMEM((NCHUNK // NS // 2, CHUNK), _i32),
        pltpu.VMEM((CHUNK, 128), _f32),
        pltpu.VMEM((CHUNK, 128), _f32),
        pltpu.VMEM_SHARED((NP, 128), _f32),
        pltpu.SemaphoreType.DMA,
        pltpu.SemaphoreType.DMA,
        pltpu.SemaphoreType.DMA,
        pltpu.SemaphoreType.DMA,
    ],
)
def _prop_kernel(g2_hbm, srccat_hbm, dst_hbm, acc2_hbm,
                 srcv, dstv, rows0, rows1, acc, semg0, semg1, sems0, sems1):
    c = lax.axis_index("c")
    s = lax.axis_index("s")
    npt = NCHUNK // NS
    nb = npt // 2
    pltpu.sync_copy(g2_hbm.at[pl.ds(c * NP + s * RPT, RPT)],
                    acc.at[pl.ds(s * RPT, RPT)])
    plsc.subcore_barrier()

    def gath(k, rows_x, semg_x):
        pltpu.async_copy(g2_hbm.at[srcv.at[k]], rows_x, semg_x)

    def gath_wait(k, rows_x, semg_x):
        pltpu.make_async_copy(g2_hbm.at[srcv.at[k]], rows_x, semg_x).wait()

    def scat(k, rows_x, sems_x):
        pltpu.async_copy(rows_x, acc.at[dstv.at[k]], sems_x, add=True)

    def scat_wait(k, rows_x, sems_x):
        pltpu.make_async_copy(rows_x, acc.at[dstv.at[k]], sems_x).wait()

    def step(k, rows_a, semg_a, sems_a, rows_b, semg_b, sems_b):
        @pl.when(k > 0)
        def _():
            scat_wait(k - 1, rows_b, sems_b)
        @pl.when(k + 1 < nb)
        def _():
            gath(k + 1, rows_b, semg_b)
        gath_wait(k, rows_a, semg_a)
        scat(k, rows_a, sems_a)

    def body(j, carry):
        step(2 * j, rows0, semg0, sems0, rows1, semg1, sems1)
        step(2 * j + 1, rows1, semg1, sems1, rows0, semg0, sems0)
        return carry

    for m in range(2):
        pltpu.sync_copy(
            srccat_hbm.at[pl.ds(c * NCHUNK + s * npt + m * nb, nb)], srcv)
        pltpu.sync_copy(dst_hbm.at[pl.ds(s * npt + m * nb, nb)], dstv)
        gath(0, rows0, semg0)
        lax.fori_loop(0, nb // 2, body, 0)
        scat_wait(nb - 1, rows1, sems1)

    plsc.subcore_barrier()
    pltpu.sync_copy(acc.at[pl.ds(s * RPT, RPT)],
                    acc2_hbm.at[pl.ds(c * NP + s * RPT, RPT)])


@functools.partial(
    pl.kernel,
    out_type=jax.ShapeDtypeStruct((NC * NP, 128), _f32),
    mesh=_MESH,
    scratch_types=[
        pltpu.VMEM((NCHUNK // (NC * NS), CHUNK), _i32),
        pltpu.VMEM((NCHUNK // (NC * NS), CHUNK), _i32),
        pltpu.VMEM((CHUNK, 128), _f32),
        pltpu.VMEM((CHUNK, 128), _f32),
        pltpu.VMEM_SHARED((NP, 128), _f32),
        pltpu.SemaphoreType.DMA,
        pltpu.SemaphoreType.DMA,
        pltpu.SemaphoreType.DMA,
        pltpu.SemaphoreType.DMA,
    ],
)
def _prop3_kernel(g_hbm, src_hbm, dst_hbm, acc2_hbm,
                  srcv, dstv, rows0, rows1, acc, semg0, semg1, sems0, sems1):
    c = lax.axis_index("c")
    s = lax.axis_index("s")
    npc = NCHUNK // (NC * NS)
    base = c * (NCHUNK // NC) + s * npc
    pltpu.sync_copy(src_hbm.at[pl.ds(base, npc)], srcv)
    pltpu.sync_copy(dst_hbm.at[pl.ds(base, npc)], dstv)
    pltpu.sync_copy(g_hbm.at[pl.ds(s * RPT, RPT)], acc.at[pl.ds(s * RPT, RPT)])
    plsc.subcore_barrier()

    def gath(k, rows_x, semg_x):
        pltpu.async_copy(g_hbm.at[srcv.at[k]], rows_x, semg_x)

    def gath_wait(k, rows_x, semg_x):
        pltpu.make_async_copy(g_hbm.at[srcv.at[k]], rows_x, semg_x).wait()

    def scat(k, rows_x, sems_x):
        pltpu.async_copy(rows_x, acc.at[dstv.at[k]], sems_x, add=True)

    def scat_wait(k, rows_x, sems_x):
        pltpu.make_async_copy(rows_x, acc.at[dstv.at[k]], sems_x).wait()

    def step(k, rows_a, semg_a, sems_a, rows_b, semg_b, sems_b):
        @pl.when(k > 0)
        def _():
            scat_wait(k - 1, rows_b, sems_b)
        @pl.when(k + 1 < npc)
        def _():
            gath(k + 1, rows_b, semg_b)
        gath_wait(k, rows_a, semg_a)
        scat(k, rows_a, sems_a)

    def body(j, carry):
        step(2 * j, rows0, semg0, sems0, rows1, semg1, sems1)
        step(2 * j + 1, rows1, semg1, sems1, rows0, semg0, sems0)
        return carry

    gath(0, rows0, semg0)
    lax.fori_loop(0, npc // 2, body, 0)
    scat_wait(npc - 1, rows1, sems1)
    plsc.subcore_barrier()
    pltpu.sync_copy(acc.at[pl.ds(s * RPT, RPT)],
                    acc2_hbm.at[pl.ds(c * NP + s * RPT, RPT)])


_BLK = 5120
_GRID = NP // _BLK

_PREC = lax.Precision.DEFAULT


def _dot(a, b):
    return lax.dot_general(a, b, (((1,), (0,)), ((), ())),
                           preferred_element_type=_f32, precision=_PREC)


def _t0a_body(x_ref, w1_ref, p_ref):
    p = _dot(x_ref[...], w1_ref[...])
    p_ref[0] = p[:, :128]
    p_ref[1] = p[:, 128:]


def _t0b_body(p_ref, dega_ref, degb_ref, disb_ref, g_ref):
    deg = dega_ref[:, 0:1] + degb_ref[:, 0:1] + 1.0
    dis = lax.rsqrt(deg)
    d = jnp.broadcast_to(dis, (_BLK, 128))
    disb_ref[...] = d
    g_ref[0] = p_ref[0] * d
    g_ref[1] = p_ref[1] * d


def _t1_body(acc_ref, disb_ref, b_ref, w_ref, g_ref):
    d = disb_ref[...]
    b = b_ref[...]
    zl = jnp.maximum(d * acc_ref[0] + b[:, :128], 0.0)
    zr = jnp.maximum(d * acc_ref[1] + b[:, 128:], 0.0)
    u = jnp.concatenate([d * zl, d * zr], axis=1)
    g = _dot(u, w_ref[...])
    g_ref[0] = g[:, :128]
    g_ref[1] = g[:, 128:]


def _t2_body(acc_ref, disb_ref, b_ref, w_ref, g3_ref):
    d = disb_ref[...]
    b = b_ref[...]
    zl = jnp.maximum(d * acc_ref[0] + b[:, :128], 0.0)
    zr = jnp.maximum(d * acc_ref[1] + b[:, 128:], 0.0)
    u = jnp.concatenate([d * zl, d * zr], axis=1)
    g3_ref[...] = _dot(u, w_ref[...])


def _t3_body(acc_ref, g3_ref, disb_ref, b_ref, out_ref):
    out_ref[...] = (disb_ref[...] * (acc_ref[0] + acc_ref[1] - g3_ref[...])
                    + b_ref[...])


def _row_spec(width):
    return pl.BlockSpec((_BLK, width), lambda i: (i, 0))


def _halves_spec():
    return pl.BlockSpec((2, _BLK, 128), lambda i: (0, i, 0))


def _full_spec(shape):
    return pl.BlockSpec(shape, lambda i: (0,) * len(shape))


def _t0a_call(x, W1):
    return pl.pallas_call(
        _t0a_body,
        grid=(_GRID,),
        in_specs=[_row_spec(256), _full_spec((256, 256))],
        out_specs=[_halves_spec()],
        out_shape=[jax.ShapeDtypeStruct((2, NP, 128), _f32)],
    )(x, W1)[0]


def _t0b_call(p, deg2):
    return pl.pallas_call(
        _t0b_body,
        grid=(_GRID,),
        in_specs=[_halves_spec(),
                  pl.BlockSpec((_BLK, DEGW), lambda i: (i, 0)),
                  pl.BlockSpec((_BLK, DEGW), lambda i: (i + NP // _BLK, 0))],
        out_specs=[_row_spec(128), _halves_spec()],
        out_shape=[jax.ShapeDtypeStruct((NP, 128), _f32),
                   jax.ShapeDtypeStruct((2, NP, 128), _f32)],
    )(p, deg2, deg2)


def _t1_call(acc2, disb, b, W):
    return pl.pallas_call(
        _t1_body,
        grid=(_GRID,),
        in_specs=[_halves_spec(), _row_spec(128),
                  _full_spec((1, 256)), _full_spec((256, 256))],
        out_specs=[_halves_spec()],
        out_shape=[jax.ShapeDtypeStruct((2, NP, 128), _f32)],
    )(acc2, disb, b, W)[0]


def _t2_call(acc2, disb, b, W):
    return pl.pallas_call(
        _t2_body,
        grid=(_GRID,),
        in_specs=[_halves_spec(), _row_spec(128),
                  _full_spec((1, 256)), _full_spec((256, 128))],
        out_specs=[_row_spec(128)],
        out_shape=[jax.ShapeDtypeStruct((NP, 128), _f32)],
    )(acc2, disb, b, W)[0]


def _t3_call(acc2, g3, disb, b):
    return pl.pallas_call(
        _t3_body,
        grid=(_GRID,),
        in_specs=[_halves_spec(), _row_spec(128), _row_spec(128),
                  _full_spec((1, 128))],
        out_specs=[_row_spec(128)],
        out_shape=[jax.ShapeDtypeStruct((N, 128), _f32)],
    )(acc2, g3, disb, b)[0]


def kernel(x, edge_index, W1, b1, W2, b2, W3, b3):
    src = edge_index[0].astype(_i32)
    dst = edge_index[1].astype(_i32)
    pad = EP - E
    api = jnp.arange(pad, dtype=_i32)
    src_p = jnp.concatenate([src, api % N])
    dst_p = jnp.concatenate([dst, N + api % (NP - N)])
    src2 = src_p.reshape(NCHUNK, CHUNK)
    dst2 = dst_p.reshape(NCHUNK, CHUNK)
    srccat = jnp.concatenate([src2, src2 + NP])

    ones_r = jnp.ones((CHUNK, DEGW), _f32)
    zdeg = jnp.zeros((NP, DEGW), _f32)
    deg2 = _deg_kernel(dst2, ones_r, zdeg)

    xp = jnp.pad(x, ((0, NP - N), (0, 0)))
    p1 = _t0a_call(xp, W1)
    disb, g1 = _t0b_call(p1, deg2)
    a1 = _prop_kernel(g1.reshape(2 * NP, 128), srccat, dst2)
    g2 = _t1_call(a1.reshape(2, NP, 128), disb, b1.reshape(1, 256), W2)
    a2 = _prop_kernel(g2.reshape(2 * NP, 128), srccat, dst2)
    g3 = _t2_call(a2.reshape(2, NP, 128), disb, b2.reshape(1, 256), W3)
    a3 = _prop3_kernel(g3, src2, dst2)
    return _t3_call(a3.reshape(2, NP, 128), g3, disb, b3.reshape(1, 128))


# speedup vs baseline: 17.7809x; 1.0032x over previous
import functools

import jax
import jax.numpy as jnp
from jax import lax
from jax.experimental import pallas as pl
from jax.experimental.pallas import tpu as pltpu
from jax.experimental.pallas import tpu_sc as plsc

N = 10000
NP = 10240
E = 160000
CHUNK = 128
EP = 163840
NCHUNK = EP // CHUNK
NC, NS = 2, 16
RPT = NP // NS
DEGW = 128

_f32 = jnp.float32
_i32 = jnp.int32

_MESH = plsc.VectorSubcoreMesh(core_axis_name="c", subcore_axis_name="s")


@functools.partial(
    pl.kernel,
    out_type=jax.ShapeDtypeStruct((NC * NP, DEGW), _f32),
    mesh=_MESH,
    scratch_types=[
        pltpu.VMEM((NCHUNK // (NC * NS), CHUNK), _i32),
        pltpu.VMEM((CHUNK, DEGW), _f32),
        pltpu.VMEM_SHARED((NP, DEGW), _f32),
        pltpu.SemaphoreType.DMA,
    ],
)
def _deg_kernel(dst_hbm, ones_hbm, zeros_hbm, deg2_hbm, dstv, onesv, table, sem):
    c = lax.axis_index("c")
    s = lax.axis_index("s")
    w = s * NC + c
    npc = NCHUNK // (NC * NS)
    grp = 8
    pltpu.sync_copy(ones_hbm, onesv)
    pltpu.sync_copy(dst_hbm.at[pl.ds(w * npc, npc)], dstv)
    pltpu.sync_copy(zeros_hbm.at[pl.ds(s * RPT, RPT)], table.at[pl.ds(s * RPT, RPT)])
    plsc.subcore_barrier()

    def body(j, carry):
        for b in range(grp):
            pltpu.async_copy(onesv, table.at[dstv.at[j * grp + b]], sem, add=True)
        for b in range(grp):
            pltpu.make_async_copy(onesv, table.at[dstv.at[j * grp + b]], sem).wait()
        return carry

    lax.fori_loop(0, npc // grp, body, 0)
    plsc.subcore_barrier()
    pltpu.sync_copy(table.at[pl.ds(s * RPT, RPT)],
                    deg2_hbm.at[pl.ds(c * NP + s * RPT, RPT)])


@functools.partial(
    pl.kernel,
    out_type=jax.ShapeDtypeStruct((NC * NP, 128), _f32),
    mesh=_MESH,
    scratch_types=[
        pltpu.VMEM((NCHUNK // NS // 2, CHUNK), _i32),
        pltpu.VMEM((NCHUNK // NS // 2, CHUNK), _i32),
        pltpu.VMEM((CHUNK, 128), _f32),
        pltpu.VMEM((CHUNK, 128), _f32),
        pltpu.VMEM_SHARED((NP, 128), _f32),
        pltpu.SemaphoreType.DMA,
        pltpu.SemaphoreType.DMA,
        pltpu.SemaphoreType.DMA,
        pltpu.SemaphoreType.DMA,
    ],
)
def _prop_kernel(g2_hbm, srccat_hbm, dst_hbm, acc2_hbm,
                 srcv, dstv, rows0, rows1, acc, semg0, semg1, sems0, sems1):
    c = lax.axis_index("c")
    s = lax.axis_index("s")
    npt = NCHUNK // NS
    nb = npt // 2
    pltpu.sync_copy(g2_hbm.at[pl.ds(c * NP + s * RPT, RPT)],
                    acc.at[pl.ds(s * RPT, RPT)])
    plsc.subcore_barrier()

    def gath(k, rows_x, semg_x):
        pltpu.async_copy(g2_hbm.at[srcv.at[k]], rows_x, semg_x)

    def gath_wait(k, rows_x, semg_x):
        pltpu.make_async_copy(g2_hbm.at[srcv.at[k]], rows_x, semg_x).wait()

    def scat(k, rows_x, sems_x):
        pltpu.async_copy(rows_x, acc.at[dstv.at[k]], sems_x, add=True)

    def scat_wait(k, rows_x, sems_x):
        pltpu.make_async_copy(rows_x, acc.at[dstv.at[k]], sems_x).wait()

    def step(k, rows_a, semg_a, sems_a, rows_b, semg_b, sems_b):
        @pl.when(k > 0)
        def _():
            scat_wait(k - 1, rows_b, sems_b)
        @pl.when(k + 1 < nb)
        def _():
            gath(k + 1, rows_b, semg_b)
        gath_wait(k, rows_a, semg_a)
        scat(k, rows_a, sems_a)

    def body(j, carry):
        step(2 * j, rows0, semg0, sems0, rows1, semg1, sems1)
        step(2 * j + 1, rows1, semg1, sems1, rows0, semg0, sems0)
        return carry

    for m in range(2):
        pltpu.sync_copy(
            srccat_hbm.at[pl.ds(c * NCHUNK + s * npt + m * nb, nb)], srcv)
        pltpu.sync_copy(dst_hbm.at[pl.ds(s * npt + m * nb, nb)], dstv)
        gath(0, rows0, semg0)
        lax.fori_loop(0, nb // 2, body, 0)
        scat_wait(nb - 1, rows1, sems1)

    plsc.subcore_barrier()
    pltpu.sync_copy(acc.at[pl.ds(s * RPT, RPT)],
                    acc2_hbm.at[pl.ds(c * NP + s * RPT, RPT)])


@functools.partial(
    pl.kernel,
    out_type=jax.ShapeDtypeStruct((NC * NP, 128), _f32),
    mesh=_MESH,
    scratch_types=[
        pltpu.VMEM((NCHUNK // (NC * NS), CHUNK), _i32),
        pltpu.VMEM((NCHUNK // (NC * NS), CHUNK), _i32),
        pltpu.VMEM((CHUNK, 128), _f32),
        pltpu.VMEM((CHUNK, 128), _f32),
        pltpu.VMEM_SHARED((NP, 128), _f32),
        pltpu.SemaphoreType.DMA,
        pltpu.SemaphoreType.DMA,
        pltpu.SemaphoreType.DMA,
        pltpu.SemaphoreType.DMA,
    ],
)
def _prop3_kernel(g_hbm, src_hbm, dst_hbm, acc2_hbm,
                  srcv, dstv, rows0, rows1, acc, semg0, semg1, sems0, sems1):
    c = lax.axis_index("c")
    s = lax.axis_index("s")
    npc = NCHUNK // (NC * NS)
    base = c * (NCHUNK // NC) + s * npc
    pltpu.sync_copy(src_hbm.at[pl.ds(base, npc)], srcv)
    pltpu.sync_copy(dst_hbm.at[pl.ds(base, npc)], dstv)
    pltpu.sync_copy(g_hbm.at[pl.ds(s * RPT, RPT)], acc.at[pl.ds(s * RPT, RPT)])
    plsc.subcore_barrier()

    def gath(k, rows_x, semg_x):
        pltpu.async_copy(g_hbm.at[srcv.at[k]], rows_x, semg_x)

    def gath_wait(k, rows_x, semg_x):
        pltpu.make_async_copy(g_hbm.at[srcv.at[k]], rows_x, semg_x).wait()

    def scat(k, rows_x, sems_x):
        pltpu.async_copy(rows_x, acc.at[dstv.at[k]], sems_x, add=True)

    def scat_wait(k, rows_x, sems_x):
        pltpu.make_async_copy(rows_x, acc.at[dstv.at[k]], sems_x).wait()

    def step(k, rows_a, semg_a, sems_a, rows_b, semg_b, sems_b):
        @pl.when(k > 0)
        def _():
            scat_wait(k - 1, rows_b, sems_b)
        @pl.when(k + 1 < npc)
        def _():
            gath(k + 1, rows_b, semg_b)
        gath_wait(k, rows_a, semg_a)
        scat(k, rows_a, sems_a)

    def body(j, carry):
        step(2 * j, rows0, semg0, sems0, rows1, semg1, sems1)
        step(2 * j + 1, rows1, semg1, sems1, rows0, semg0, sems0)
        return carry

    gath(0, rows0, semg0)
    lax.fori_loop(0, npc // 2, body, 0)
    scat_wait(npc - 1, rows1, sems1)
    plsc.subcore_barrier()
    pltpu.sync_copy(acc.at[pl.ds(s * RPT, RPT)],
                    acc2_hbm.at[pl.ds(c * NP + s * RPT, RPT)])


_BLK = 5120
_GRID = NP // _BLK

_PREC = lax.Precision.DEFAULT


def _dot(a, b):
    return lax.dot_general(a, b, (((1,), (0,)), ((), ())),
                           preferred_element_type=_f32, precision=_PREC)


def _t0_body(x_ref, w1_ref, dega_ref, degb_ref, disb_ref, g_ref):
    deg = dega_ref[:, 0:1] + degb_ref[:, 0:1] + 1.0
    dis = lax.rsqrt(deg)
    disb_ref[...] = jnp.broadcast_to(dis, (_BLK, 128))
    u = (x_ref[...] * dis).astype(jnp.bfloat16)
    g = _dot(u, w1_ref[...])
    g_ref[0] = g[:, :128]
    g_ref[1] = g[:, 128:]


def _t1_body(acc_ref, disb_ref, b_ref, w_ref, g_ref):
    d = disb_ref[...]
    b = b_ref[...]
    zl = jnp.maximum(d * acc_ref[0] + b[:, :128], 0.0)
    zr = jnp.maximum(d * acc_ref[1] + b[:, 128:], 0.0)
    u = jnp.concatenate([d * zl, d * zr], axis=1).astype(jnp.bfloat16)
    g = _dot(u, w_ref[...])
    g_ref[0] = g[:, :128]
    g_ref[1] = g[:, 128:]


def _t2_body(acc_ref, disb_ref, b_ref, w_ref, g3_ref):
    d = disb_ref[...]
    b = b_ref[...]
    zl = jnp.maximum(d * acc_ref[0] + b[:, :128], 0.0)
    zr = jnp.maximum(d * acc_ref[1] + b[:, 128:], 0.0)
    u = jnp.concatenate([d * zl, d * zr], axis=1).astype(jnp.bfloat16)
    g3_ref[...] = _dot(u, w_ref[...])


def _t3_body(acc_ref, g3_ref, disb_ref, b_ref, out_ref):
    out_ref[...] = (disb_ref[...] * (acc_ref[0] + acc_ref[1] - g3_ref[...])
                    + b_ref[...])


def _row_spec(width):
    return pl.BlockSpec((_BLK, width), lambda i: (i, 0))


def _halves_spec():
    return pl.BlockSpec((2, _BLK, 128), lambda i: (0, i, 0))


def _full_spec(shape):
    return pl.BlockSpec(shape, lambda i: (0,) * len(shape))


def _t0_call(x, W1, deg2):
    return pl.pallas_call(
        _t0_body,
        grid=(_GRID,),
        in_specs=[_row_spec(256), _full_spec((256, 256)),
                  pl.BlockSpec((_BLK, DEGW), lambda i: (i, 0)),
                  pl.BlockSpec((_BLK, DEGW), lambda i: (i + NP // _BLK, 0))],
        out_specs=[_row_spec(128), _halves_spec()],
        out_shape=[jax.ShapeDtypeStruct((NP, 128), _f32),
                   jax.ShapeDtypeStruct((2, NP, 128), _f32)],
    )(x, W1, deg2, deg2)


def _t1_call(acc2, disb, b, W):
    return pl.pallas_call(
        _t1_body,
        grid=(_GRID,),
        in_specs=[_halves_spec(), _row_spec(128),
                  _full_spec((1, 256)), _full_spec((256, 256))],
        out_specs=[_halves_spec()],
        out_shape=[jax.ShapeDtypeStruct((2, NP, 128), _f32)],
    )(acc2, disb, b, W)[0]


def _t2_call(acc2, disb, b, W):
    return pl.pallas_call(
        _t2_body,
        grid=(_GRID,),
        in_specs=[_halves_spec(), _row_spec(128),
                  _full_spec((1, 256)), _full_spec((256, 128))],
        out_specs=[_row_spec(128)],
        out_shape=[jax.ShapeDtypeStruct((NP, 128), _f32)],
    )(acc2, disb, b, W)[0]


def _t3_call(acc2, g3, disb, b):
    return pl.pallas_call(
        _t3_body,
        grid=(_GRID,),
        in_specs=[_halves_spec(), _row_spec(128), _row_spec(128),
                  _full_spec((1, 128))],
        out_specs=[_row_spec(128)],
        out_shape=[jax.ShapeDtypeStruct((N, 128), _f32)],
    )(acc2, g3, disb, b)[0]


def kernel(x, edge_index, W1, b1, W2, b2, W3, b3):
    src = edge_index[0].astype(_i32)
    dst = edge_index[1].astype(_i32)
    pad = EP - E
    api = jnp.arange(pad, dtype=_i32)
    src_p = jnp.concatenate([src, api % N])
    dst_p = jnp.concatenate([dst, N + api % (NP - N)])
    src2 = src_p.reshape(NCHUNK, CHUNK)
    dst2 = dst_p.reshape(NCHUNK, CHUNK)
    srccat = jnp.concatenate([src2, src2 + NP])

    ones_r = jnp.ones((CHUNK, DEGW), _f32)
    zdeg = jnp.zeros((NP, DEGW), _f32)
    deg2 = _deg_kernel(dst2, ones_r, zdeg)

    xp = jnp.pad(x, ((0, NP - N), (0, 0)))
    disb, g1 = _t0_call(xp, W1.astype(jnp.bfloat16), deg2)
    a1 = _prop_kernel(g1.reshape(2 * NP, 128), srccat, dst2)
    W2b = W2.astype(jnp.bfloat16)
    g2 = _t1_call(a1.reshape(2, NP, 128), disb, b1.reshape(1, 256), W2b)
    a2 = _prop_kernel(g2.reshape(2 * NP, 128), srccat, dst2)
    g3 = _t2_call(a2.reshape(2, NP, 128), disb, b2.reshape(1, 256), W3.astype(jnp.bfloat16))
    a3 = _prop3_kernel(g3, src2, dst2)
    return _t3_call(a3.reshape(2, NP, 128), g3, disb, b3.reshape(1, 128))


# speedup vs baseline: 17.9655x; 1.0104x over previous
import functools

import jax
import jax.numpy as jnp
from jax import lax
from jax.experimental import pallas as pl
from jax.experimental.pallas import tpu as pltpu
from jax.experimental.pallas import tpu_sc as plsc

N = 10000
NP = 10240
E = 160000
CHUNK = 128
EP = 163840
NCHUNK = EP // CHUNK
NC, NS = 2, 16
RPT = NP // NS
DEGW = 128

_f32 = jnp.float32
_i32 = jnp.int32

_MESH = plsc.VectorSubcoreMesh(core_axis_name="c", subcore_axis_name="s")


@functools.partial(
    pl.kernel,
    out_type=jax.ShapeDtypeStruct((NC * NP, DEGW), _f32),
    mesh=_MESH,
    scratch_types=[
        pltpu.VMEM((NCHUNK // (NC * NS), CHUNK), _i32),
        pltpu.VMEM((CHUNK, DEGW), _f32),
        pltpu.VMEM_SHARED((NP, DEGW), _f32),
        pltpu.SemaphoreType.DMA,
    ],
)
def _deg_kernel(dst_hbm, ones_hbm, zeros_hbm, deg2_hbm, dstv, onesv, table, sem):
    c = lax.axis_index("c")
    s = lax.axis_index("s")
    w = s * NC + c
    npc = NCHUNK // (NC * NS)
    grp = 8
    pltpu.sync_copy(ones_hbm, onesv)
    pltpu.sync_copy(dst_hbm.at[pl.ds(w * npc, npc)], dstv)
    pltpu.sync_copy(zeros_hbm.at[pl.ds(s * RPT, RPT)], table.at[pl.ds(s * RPT, RPT)])
    plsc.subcore_barrier()

    def body(j, carry):
        for b in range(grp):
            pltpu.async_copy(onesv, table.at[dstv.at[j * grp + b]], sem, add=True)
        for b in range(grp):
            pltpu.make_async_copy(onesv, table.at[dstv.at[j * grp + b]], sem).wait()
        return carry

    lax.fori_loop(0, npc // grp, body, 0)
    plsc.subcore_barrier()
    pltpu.sync_copy(table.at[pl.ds(s * RPT, RPT)],
                    deg2_hbm.at[pl.ds(c * NP + s * RPT, RPT)])


@functools.partial(
    pl.kernel,
    out_type=jax.ShapeDtypeStruct((NC * NP, 128), _f32),
    mesh=_MESH,
    scratch_types=[
        pltpu.VMEM((NCHUNK // NS // 2, CHUNK), _i32),
        pltpu.VMEM((NCHUNK // NS // 2, CHUNK), _i32),
        pltpu.VMEM((CHUNK, 128), _f32),
        pltpu.VMEM((CHUNK, 128), _f32),
        pltpu.VMEM_SHARED((NP, 128), _f32),
        pltpu.SemaphoreType.DMA,
        pltpu.SemaphoreType.DMA,
        pltpu.SemaphoreType.DMA,
        pltpu.SemaphoreType.DMA,
    ],
)
def _prop_kernel(g2_hbm, srccat_hbm, dst_hbm, acc2_hbm,
                 srcv, dstv, rows0, rows1, acc, semg0, semg1, sems0, sems1):
    c = lax.axis_index("c")
    s = lax.axis_index("s")
    npt = NCHUNK // NS
    nb = npt // 2
    pltpu.sync_copy(g2_hbm.at[pl.ds(c * NP + s * RPT, RPT)],
                    acc.at[pl.ds(s * RPT, RPT)])
    plsc.subcore_barrier()

    def gath(k, rows_x, semg_x):
        pltpu.async_copy(g2_hbm.at[srcv.at[k]], rows_x, semg_x)

    def gath_wait(k, rows_x, semg_x):
        pltpu.make_async_copy(g2_hbm.at[srcv.at[k]], rows_x, semg_x).wait()

    def scat(k, rows_x, sems_x):
        pltpu.async_copy(rows_x, acc.at[dstv.at[k]], sems_x, add=True)

    def scat_wait(k, rows_x, sems_x):
        pltpu.make_async_copy(rows_x, acc.at[dstv.at[k]], sems_x).wait()

    def step(k, rows_a, semg_a, sems_a, rows_b, semg_b, sems_b):
        @pl.when(k > 0)
        def _():
            scat_wait(k - 1, rows_b, sems_b)
        @pl.when(k + 1 < nb)
        def _():
            gath(k + 1, rows_b, semg_b)
        gath_wait(k, rows_a, semg_a)
        scat(k, rows_a, sems_a)

    def body(j, carry):
        step(2 * j, rows0, semg0, sems0, rows1, semg1, sems1)
        step(2 * j + 1, rows1, semg1, sems1, rows0, semg0, sems0)
        return carry

    for m in range(2):
        pltpu.sync_copy(
            srccat_hbm.at[pl.ds(c * NCHUNK + s * npt + m * nb, nb)], srcv)
        pltpu.sync_copy(dst_hbm.at[pl.ds(s * npt + m * nb, nb)], dstv)
        gath(0, rows0, semg0)
        lax.fori_loop(0, nb // 2, body, 0)
        scat_wait(nb - 1, rows1, sems1)

    plsc.subcore_barrier()
    pltpu.sync_copy(acc.at[pl.ds(s * RPT, RPT)],
                    acc2_hbm.at[pl.ds(c * NP + s * RPT, RPT)])


@functools.partial(
    pl.kernel,
    out_type=jax.ShapeDtypeStruct((NC * NP, 128), _f32),
    mesh=_MESH,
    scratch_types=[
        pltpu.VMEM((NCHUNK // (NC * NS), CHUNK), _i32),
        pltpu.VMEM((NCHUNK // (NC * NS), CHUNK), _i32),
        pltpu.VMEM((CHUNK, 128), _f32),
        pltpu.VMEM((CHUNK, 128), _f32),
        pltpu.VMEM_SHARED((NP, 128), _f32),
        pltpu.SemaphoreType.DMA,
        pltpu.SemaphoreType.DMA,
        pltpu.SemaphoreType.DMA,
        pltpu.SemaphoreType.DMA,
    ],
)
def _prop3_kernel(g_hbm, src_hbm, dst_hbm, acc2_hbm,
                  srcv, dstv, rows0, rows1, acc, semg0, semg1, sems0, sems1):
    c = lax.axis_index("c")
    s = lax.axis_index("s")
    npc = NCHUNK // (NC * NS)
    base = c * (NCHUNK // NC) + s * npc
    pltpu.sync_copy(src_hbm.at[pl.ds(base, npc)], srcv)
    pltpu.sync_copy(dst_hbm.at[pl.ds(base, npc)], dstv)
    pltpu.sync_copy(g_hbm.at[pl.ds(s * RPT, RPT)], acc.at[pl.ds(s * RPT, RPT)])
    plsc.subcore_barrier()

    def gath(k, rows_x, semg_x):
        pltpu.async_copy(g_hbm.at[srcv.at[k]], rows_x, semg_x)

    def gath_wait(k, rows_x, semg_x):
        pltpu.make_async_copy(g_hbm.at[srcv.at[k]], rows_x, semg_x).wait()

    def scat(k, rows_x, sems_x):
        pltpu.async_copy(rows_x, acc.at[dstv.at[k]], sems_x, add=True)

    def scat_wait(k, rows_x, sems_x):
        pltpu.make_async_copy(rows_x, acc.at[dstv.at[k]], sems_x).wait()

    def step(k, rows_a, semg_a, sems_a, rows_b, semg_b, sems_b):
        @pl.when(k > 0)
        def _():
            scat_wait(k - 1, rows_b, sems_b)
        @pl.when(k + 1 < npc)
        def _():
            gath(k + 1, rows_b, semg_b)
        gath_wait(k, rows_a, semg_a)
        scat(k, rows_a, sems_a)

    def body(j, carry):
        step(2 * j, rows0, semg0, sems0, rows1, semg1, sems1)
        step(2 * j + 1, rows1, semg1, sems1, rows0, semg0, sems0)
        return carry

    gath(0, rows0, semg0)
    lax.fori_loop(0, npc // 2, body, 0)
    scat_wait(npc - 1, rows1, sems1)
    plsc.subcore_barrier()
    pltpu.sync_copy(acc.at[pl.ds(s * RPT, RPT)],
                    acc2_hbm.at[pl.ds(c * NP + s * RPT, RPT)])


_BLK = 5120
_GRID = NP // _BLK

_PREC = lax.Precision.DEFAULT


def _dot(a, b):
    return lax.dot_general(a, b, (((1,), (0,)), ((), ())),
                           preferred_element_type=_f32, precision=_PREC)


def _t0_body(x_ref, w1_ref, dega_ref, degb_ref, disb_ref, g_ref):
    deg = dega_ref[:, 0:1] + degb_ref[:, 0:1] + 1.0
    dis = lax.rsqrt(deg)
    disb_ref[...] = jnp.broadcast_to(dis, (_BLK, 128))
    u = (x_ref[...] * dis).astype(jnp.bfloat16)
    g = _dot(u, w1_ref[...])
    g_ref[0] = g[:, :128]
    g_ref[1] = g[:, 128:]


def _t1_body(acc_ref, disb_ref, b_ref, w_ref, g_ref):
    d = disb_ref[...]
    b = b_ref[...]
    zl = jnp.maximum(d * acc_ref[0] + b[:, :128], 0.0)
    zr = jnp.maximum(d * acc_ref[1] + b[:, 128:], 0.0)
    u = jnp.concatenate([d * zl, d * zr], axis=1).astype(jnp.bfloat16)
    g = _dot(u, w_ref[...])
    g_ref[0] = g[:, :128]
    g_ref[1] = g[:, 128:]


def _t2_body(acc_ref, disb_ref, b_ref, w_ref, g3_ref):
    d = disb_ref[...]
    b = b_ref[...]
    zl = jnp.maximum(d * acc_ref[0] + b[:, :128], 0.0)
    zr = jnp.maximum(d * acc_ref[1] + b[:, 128:], 0.0)
    u = jnp.concatenate([d * zl, d * zr], axis=1).astype(jnp.bfloat16)
    g3_ref[...] = _dot(u, w_ref[...])


def _t3_body(acc_ref, g3_ref, disb_ref, b_ref, out_ref):
    out_ref[...] = (disb_ref[...] * (acc_ref[0] + acc_ref[1] - g3_ref[...])
                    + b_ref[...])


def _row_spec(width):
    return pl.BlockSpec((_BLK, width), lambda i: (i, 0))


def _halves_spec():
    return pl.BlockSpec((2, _BLK, 128), lambda i: (0, i, 0))


def _full_spec(shape):
    return pl.BlockSpec(shape, lambda i: (0,) * len(shape))


def _t0_call(x, W1, deg2):
    return pl.pallas_call(
        _t0_body,
        grid=(_GRID,),
        in_specs=[_row_spec(256), _full_spec((256, 256)),
                  pl.BlockSpec((_BLK, DEGW), lambda i: (i, 0)),
                  pl.BlockSpec((_BLK, DEGW), lambda i: (i + NP // _BLK, 0))],
        out_specs=[_row_spec(128), _halves_spec()],
        out_shape=[jax.ShapeDtypeStruct((NP, 128), _f32),
                   jax.ShapeDtypeStruct((2, NP, 128), _f32)],
    )(x, W1, deg2, deg2)


def _t1_call(acc2, disb, b, W):
    return pl.pallas_call(
        _t1_body,
        grid=(_GRID,),
        in_specs=[_halves_spec(), _row_spec(128),
                  _full_spec((1, 256)), _full_spec((256, 256))],
        out_specs=[_halves_spec()],
        out_shape=[jax.ShapeDtypeStruct((2, NP, 128), _f32)],
    )(acc2, disb, b, W)[0]


def _t2_call(acc2, disb, b, W):
    return pl.pallas_call(
        _t2_body,
        grid=(_GRID,),
        in_specs=[_halves_spec(), _row_spec(128),
                  _full_spec((1, 256)), _full_spec((256, 128))],
        out_specs=[_row_spec(128)],
        out_shape=[jax.ShapeDtypeStruct((NP, 128), _f32)],
    )(acc2, disb, b, W)[0]


def _t3_call(acc2, g3, disb, b):
    return pl.pallas_call(
        _t3_body,
        grid=(_GRID,),
        in_specs=[_halves_spec(), _row_spec(128), _row_spec(128),
                  _full_spec((1, 128))],
        out_specs=[_row_spec(128)],
        out_shape=[jax.ShapeDtypeStruct((N, 128), _f32)],
    )(acc2, g3, disb, b)[0]


def kernel(x, edge_index, W1, b1, W2, b2, W3, b3):
    src = edge_index[0].astype(_i32)
    dst = edge_index[1].astype(_i32)
    pad = EP - E
    api = jnp.arange(pad, dtype=_i32)
    src_p = jnp.concatenate([src, api % N])
    dst_p = jnp.concatenate([dst, N + api % (NP - N)])
    src2 = src_p.reshape(NCHUNK, CHUNK)
    dst2 = dst_p.reshape(NCHUNK, CHUNK)
    srccat = jnp.concatenate([src2, src2 + NP])

    ones_r = jnp.ones((CHUNK, DEGW), _f32)
    zdeg = jnp.zeros((NP, DEGW), _f32)
    deg2 = _deg_kernel(dst2, ones_r, zdeg)

    disb, g1 = _t0_call(x, W1.astype(jnp.bfloat16), deg2)
    a1 = _prop_kernel(g1.reshape(2 * NP, 128), srccat, dst2)
    W2b = W2.astype(jnp.bfloat16)
    g2 = _t1_call(a1.reshape(2, NP, 128), disb, b1.reshape(1, 256), W2b)
    a2 = _prop_kernel(g2.reshape(2 * NP, 128), srccat, dst2)
    g3 = _t2_call(a2.reshape(2, NP, 128), disb, b2.reshape(1, 256), W3.astype(jnp.bfloat16))
    a3 = _prop3_kernel(g3, src2, dst2)
    return _t3_call(a3.reshape(2, NP, 128), g3, disb, b3.reshape(1, 128))


# speedup vs baseline: 18.4175x; 1.0252x over previous
import functools

import jax
import jax.numpy as jnp
from jax import lax
from jax.experimental import pallas as pl
from jax.experimental.pallas import tpu as pltpu
from jax.experimental.pallas import tpu_sc as plsc

N = 10000
NP = 10240
E = 160000
CHUNK = 128
EP = 163840
NCHUNK = EP // CHUNK
NC, NS = 2, 16
RPT = NP // NS
DEGW = 128

_f32 = jnp.float32
_i32 = jnp.int32

_MESH = plsc.VectorSubcoreMesh(core_axis_name="c", subcore_axis_name="s")


@functools.partial(
    pl.kernel,
    out_type=jax.ShapeDtypeStruct((NC * NP, DEGW), _f32),
    mesh=_MESH,
    scratch_types=[
        pltpu.VMEM((NCHUNK // (NC * NS), CHUNK), _i32),
        pltpu.VMEM((CHUNK, DEGW), _f32),
        pltpu.VMEM((CHUNK, DEGW), _f32),
        pltpu.VMEM_SHARED((NP, DEGW), _f32),
        pltpu.SemaphoreType.DMA,
    ],
)
def _deg_kernel(dst_hbm, deg2_hbm, dstv, onesv, zerov, table, sem):
    c = lax.axis_index("c")
    s = lax.axis_index("s")
    w = s * NC + c
    npc = NCHUNK // (NC * NS)
    grp = 8
    pltpu.sync_copy(dst_hbm.at[pl.ds(w * npc, npc)], dstv)

    one16 = jnp.ones((16,), _f32)
    zero16 = jnp.zeros((16,), _f32)

    def fill(r, carry):
        for k in range(DEGW // 16):
            onesv[r, pl.ds(k * 16, 16)] = one16
            zerov[r, pl.ds(k * 16, 16)] = zero16
        return carry

    lax.fori_loop(0, CHUNK, fill, 0)
    for j in range(RPT // CHUNK):
        pltpu.sync_copy(zerov, table.at[pl.ds(s * RPT + j * CHUNK, CHUNK)])
    plsc.subcore_barrier()

    def body(j, carry):
        for b in range(grp):
            pltpu.async_copy(onesv, table.at[dstv.at[j * grp + b]], sem, add=True)
        for b in range(grp):
            pltpu.make_async_copy(onesv, table.at[dstv.at[j * grp + b]], sem).wait()
        return carry

    lax.fori_loop(0, npc // grp, body, 0)
    plsc.subcore_barrier()
    pltpu.sync_copy(table.at[pl.ds(s * RPT, RPT)],
                    deg2_hbm.at[pl.ds(c * NP + s * RPT, RPT)])


@functools.partial(
    pl.kernel,
    out_type=jax.ShapeDtypeStruct((NC * NP, 128), _f32),
    mesh=_MESH,
    scratch_types=[
        pltpu.VMEM((NCHUNK // NS // 2, CHUNK), _i32),
        pltpu.VMEM((NCHUNK // NS // 2, CHUNK), _i32),
        pltpu.VMEM((CHUNK, 128), _f32),
        pltpu.VMEM((CHUNK, 128), _f32),
        pltpu.VMEM_SHARED((NP, 128), _f32),
        pltpu.SemaphoreType.DMA,
        pltpu.SemaphoreType.DMA,
        pltpu.SemaphoreType.DMA,
        pltpu.SemaphoreType.DMA,
    ],
)
def _prop_kernel(g2_hbm, srccat_hbm, dst_hbm, acc2_hbm,
                 srcv, dstv, rows0, rows1, acc, semg0, semg1, sems0, sems1):
    c = lax.axis_index("c")
    s = lax.axis_index("s")
    npt = NCHUNK // NS
    nb = npt // 2
    pltpu.sync_copy(g2_hbm.at[pl.ds(c * NP + s * RPT, RPT)],
                    acc.at[pl.ds(s * RPT, RPT)])
    plsc.subcore_barrier()

    def gath(k, rows_x, semg_x):
        pltpu.async_copy(g2_hbm.at[srcv.at[k]], rows_x, semg_x)

    def gath_wait(k, rows_x, semg_x):
        pltpu.make_async_copy(g2_hbm.at[srcv.at[k]], rows_x, semg_x).wait()

    def scat(k, rows_x, sems_x):
        pltpu.async_copy(rows_x, acc.at[dstv.at[k]], sems_x, add=True)

    def scat_wait(k, rows_x, sems_x):
        pltpu.make_async_copy(rows_x, acc.at[dstv.at[k]], sems_x).wait()

    def step(k, rows_a, semg_a, sems_a, rows_b, semg_b, sems_b):
        @pl.when(k > 0)
        def _():
            scat_wait(k - 1, rows_b, sems_b)
        @pl.when(k + 1 < nb)
        def _():
            gath(k + 1, rows_b, semg_b)
        gath_wait(k, rows_a, semg_a)
        scat(k, rows_a, sems_a)

    def body(j, carry):
        step(2 * j, rows0, semg0, sems0, rows1, semg1, sems1)
        step(2 * j + 1, rows1, semg1, sems1, rows0, semg0, sems0)
        return carry

    for m in range(2):
        pltpu.sync_copy(
            srccat_hbm.at[pl.ds(c * NCHUNK + s * npt + m * nb, nb)], srcv)
        pltpu.sync_copy(dst_hbm.at[pl.ds(s * npt + m * nb, nb)], dstv)
        gath(0, rows0, semg0)
        lax.fori_loop(0, nb // 2, body, 0)
        scat_wait(nb - 1, rows1, sems1)

    plsc.subcore_barrier()
    pltpu.sync_copy(acc.at[pl.ds(s * RPT, RPT)],
                    acc2_hbm.at[pl.ds(c * NP + s * RPT, RPT)])


@functools.partial(
    pl.kernel,
    out_type=jax.ShapeDtypeStruct((NC * NP, 128), _f32),
    mesh=_MESH,
    scratch_types=[
        pltpu.VMEM((NCHUNK // (NC * NS), CHUNK), _i32),
        pltpu.VMEM((NCHUNK // (NC * NS), CHUNK), _i32),
        pltpu.VMEM((CHUNK, 128), _f32),
        pltpu.VMEM((CHUNK, 128), _f32),
        pltpu.VMEM_SHARED((NP, 128), _f32),
        pltpu.SemaphoreType.DMA,
        pltpu.SemaphoreType.DMA,
        pltpu.SemaphoreType.DMA,
        pltpu.SemaphoreType.DMA,
    ],
)
def _prop3_kernel(g_hbm, src_hbm, dst_hbm, acc2_hbm,
                  srcv, dstv, rows0, rows1, acc, semg0, semg1, sems0, sems1):
    c = lax.axis_index("c")
    s = lax.axis_index("s")
    npc = NCHUNK // (NC * NS)
    base = c * (NCHUNK // NC) + s * npc
    pltpu.sync_copy(src_hbm.at[pl.ds(base, npc)], srcv)
    pltpu.sync_copy(dst_hbm.at[pl.ds(base, npc)], dstv)
    pltpu.sync_copy(g_hbm.at[pl.ds(s * RPT, RPT)], acc.at[pl.ds(s * RPT, RPT)])
    plsc.subcore_barrier()

    def gath(k, rows_x, semg_x):
        pltpu.async_copy(g_hbm.at[srcv.at[k]], rows_x, semg_x)

    def gath_wait(k, rows_x, semg_x):
        pltpu.make_async_copy(g_hbm.at[srcv.at[k]], rows_x, semg_x).wait()

    def scat(k, rows_x, sems_x):
        pltpu.async_copy(rows_x, acc.at[dstv.at[k]], sems_x, add=True)

    def scat_wait(k, rows_x, sems_x):
        pltpu.make_async_copy(rows_x, acc.at[dstv.at[k]], sems_x).wait()

    def step(k, rows_a, semg_a, sems_a, rows_b, semg_b, sems_b):
        @pl.when(k > 0)
        def _():
            scat_wait(k - 1, rows_b, sems_b)
        @pl.when(k + 1 < npc)
        def _():
            gath(k + 1, rows_b, semg_b)
        gath_wait(k, rows_a, semg_a)
        scat(k, rows_a, sems_a)

    def body(j, carry):
        step(2 * j, rows0, semg0, sems0, rows1, semg1, sems1)
        step(2 * j + 1, rows1, semg1, sems1, rows0, semg0, sems0)
        return carry

    gath(0, rows0, semg0)
    lax.fori_loop(0, npc // 2, body, 0)
    scat_wait(npc - 1, rows1, sems1)
    plsc.subcore_barrier()
    pltpu.sync_copy(acc.at[pl.ds(s * RPT, RPT)],
                    acc2_hbm.at[pl.ds(c * NP + s * RPT, RPT)])


_BLK = 5120
_GRID = NP // _BLK

_PREC = lax.Precision.DEFAULT


def _dot(a, b):
    return lax.dot_general(a, b, (((1,), (0,)), ((), ())),
                           preferred_element_type=_f32, precision=_PREC)


def _t0_body(x_ref, w1_ref, dega_ref, degb_ref, disb_ref, g_ref):
    deg = dega_ref[:, 0:1] + degb_ref[:, 0:1] + 1.0
    dis = lax.rsqrt(deg)
    disb_ref[...] = jnp.broadcast_to(dis, (_BLK, 128))
    u = (x_ref[...] * dis).astype(jnp.bfloat16)
    g = _dot(u, w1_ref[...])
    g_ref[0] = g[:, :128]
    g_ref[1] = g[:, 128:]


def _t1_body(acc_ref, disb_ref, b_ref, w_ref, g_ref):
    d = disb_ref[...]
    b = b_ref[...]
    zl = jnp.maximum(d * acc_ref[0] + b[:, :128], 0.0)
    zr = jnp.maximum(d * acc_ref[1] + b[:, 128:], 0.0)
    u = jnp.concatenate([d * zl, d * zr], axis=1).astype(jnp.bfloat16)
    g = _dot(u, w_ref[...])
    g_ref[0] = g[:, :128]
    g_ref[1] = g[:, 128:]


def _t2_body(acc_ref, disb_ref, b_ref, w_ref, g3_ref):
    d = disb_ref[...]
    b = b_ref[...]
    zl = jnp.maximum(d * acc_ref[0] + b[:, :128], 0.0)
    zr = jnp.maximum(d * acc_ref[1] + b[:, 128:], 0.0)
    u = jnp.concatenate([d * zl, d * zr], axis=1).astype(jnp.bfloat16)
    g3_ref[...] = _dot(u, w_ref[...])


def _t3_body(acc_ref, g3_ref, disb_ref, b_ref, out_ref):
    out_ref[...] = (disb_ref[...] * (acc_ref[0] + acc_ref[1] - g3_ref[...])
                    + b_ref[...])


def _row_spec(width):
    return pl.BlockSpec((_BLK, width), lambda i: (i, 0))


def _halves_spec():
    return pl.BlockSpec((2, _BLK, 128), lambda i: (0, i, 0))


def _full_spec(shape):
    return pl.BlockSpec(shape, lambda i: (0,) * len(shape))


def _t0_call(x, W1, deg2):
    return pl.pallas_call(
        _t0_body,
        grid=(_GRID,),
        in_specs=[_row_spec(256), _full_spec((256, 256)),
                  pl.BlockSpec((_BLK, DEGW), lambda i: (i, 0)),
                  pl.BlockSpec((_BLK, DEGW), lambda i: (i + NP // _BLK, 0))],
        out_specs=[_row_spec(128), _halves_spec()],
        out_shape=[jax.ShapeDtypeStruct((NP, 128), _f32),
                   jax.ShapeDtypeStruct((2, NP, 128), _f32)],
    )(x, W1, deg2, deg2)


def _t1_call(acc2, disb, b, W):
    return pl.pallas_call(
        _t1_body,
        grid=(_GRID,),
        in_specs=[_halves_spec(), _row_spec(128),
                  _full_spec((1, 256)), _full_spec((256, 256))],
        out_specs=[_halves_spec()],
        out_shape=[jax.ShapeDtypeStruct((2, NP, 128), _f32)],
    )(acc2, disb, b, W)[0]


def _t2_call(acc2, disb, b, W):
    return pl.pallas_call(
        _t2_body,
        grid=(_GRID,),
        in_specs=[_halves_spec(), _row_spec(128),
                  _full_spec((1, 256)), _full_spec((256, 128))],
        out_specs=[_row_spec(128)],
        out_shape=[jax.ShapeDtypeStruct((NP, 128), _f32)],
    )(acc2, disb, b, W)[0]


def _t3_call(acc2, g3, disb, b):
    return pl.pallas_call(
        _t3_body,
        grid=(_GRID,),
        in_specs=[_halves_spec(), _row_spec(128), _row_spec(128),
                  _full_spec((1, 128))],
        out_specs=[_row_spec(128)],
        out_shape=[jax.ShapeDtypeStruct((N, 128), _f32)],
    )(acc2, g3, disb, b)[0]


def kernel(x, edge_index, W1, b1, W2, b2, W3, b3):
    src = edge_index[0].astype(_i32)
    dst = edge_index[1].astype(_i32)
    pad = EP - E
    api = jnp.arange(pad, dtype=_i32)
    src_p = jnp.concatenate([src, api % N])
    dst_p = jnp.concatenate([dst, N + api % (NP - N)])
    src2 = src_p.reshape(NCHUNK, CHUNK)
    dst2 = dst_p.reshape(NCHUNK, CHUNK)
    srccat = jnp.concatenate([src2, src2 + NP])

    deg2 = _deg_kernel(dst2)

    disb, g1 = _t0_call(x, W1.astype(jnp.bfloat16), deg2)
    a1 = _prop_kernel(g1.reshape(2 * NP, 128), srccat, dst2)
    W2b = W2.astype(jnp.bfloat16)
    g2 = _t1_call(a1.reshape(2, NP, 128), disb, b1.reshape(1, 256), W2b)
    a2 = _prop_kernel(g2.reshape(2 * NP, 128), srccat, dst2)
    g3 = _t2_call(a2.reshape(2, NP, 128), disb, b2.reshape(1, 256), W3.astype(jnp.bfloat16))
    a3 = _prop3_kernel(g3, src2, dst2)
    return _t3_call(a3.reshape(2, NP, 128), g3, disb, b3.reshape(1, 128))
